# Initial kernel scaffold; baseline (speedup 1.0000x reference)
import jax
import jax.numpy as jnp
from jax.experimental import pallas as pl


def kernel(kg_name_embed, eer_adj_index, eer_adj_data, r_head, r_tail, kg_name_w, kg_name_b, w_R_Left, w_R_Right, w_atten_r):
    raise NotImplementedError("write your pallas kernel here")


# speedup vs baseline: 1.7176x; 1.7176x over previous
import functools

import jax
import jax.numpy as jnp
from jax import lax
from jax.experimental import pallas as pl
from jax.experimental.pallas import tpu as pltpu

KG_E = 10000
KG_R = 1000
E_DIM = 128
N_EDGES = 320000
ALPHA = 0.2
BETA1 = 0.3
EXT = 144


def _ext_cols(n):
    return jnp.where(lax.broadcasted_iota(jnp.int32, (n, 16), 1) == 0, 1.0, 0.0)


def _k1_body(x_ref, w_ref, b_ref, wl_ref, wr_ref, o_ref, e_ref, l_ref, r_ref):
    y = jnp.dot(x_ref[...], w_ref[...], preferred_element_type=jnp.float32)
    y = y + b_ref[...]
    o_ref[...] = y
    e_ref[...] = jnp.concatenate([y, _ext_cols(y.shape[0])], axis=1)
    l_ref[...] = jnp.dot(y, wl_ref[...], preferred_element_type=jnp.float32)
    r_ref[...] = jnp.dot(y, wr_ref[...], preferred_element_type=jnp.float32)


def _k1(kg, w, b2, wl, wr):
    blk = 1000
    grid = KG_E // blk
    return pl.pallas_call(
        _k1_body,
        grid=(grid,),
        in_specs=[
            pl.BlockSpec((blk, 300), lambda i: (i, 0)),
            pl.BlockSpec((300, E_DIM), lambda i: (0, 0)),
            pl.BlockSpec((1, E_DIM), lambda i: (0, 0)),
            pl.BlockSpec((E_DIM, E_DIM), lambda i: (0, 0)),
            pl.BlockSpec((E_DIM, E_DIM), lambda i: (0, 0)),
        ],
        out_specs=[
            pl.BlockSpec((blk, E_DIM), lambda i: (i, 0)),
            pl.BlockSpec((blk, EXT), lambda i: (i, 0)),
            pl.BlockSpec((blk, E_DIM), lambda i: (i, 0)),
            pl.BlockSpec((blk, E_DIM), lambda i: (i, 0)),
        ],
        out_shape=[
            jax.ShapeDtypeStruct((KG_E, E_DIM), jnp.float32),
            jax.ShapeDtypeStruct((KG_E, EXT), jnp.float32),
            jax.ShapeDtypeStruct((KG_E, E_DIM), jnp.float32),
            jax.ShapeDtypeStruct((KG_E, E_DIM), jnp.float32),
        ],
    )(kg, w, b2, wl, wr)


def _ku_body(rh_ref, rt_ref, le_ref, re_ref, al_ref, ar_ref, u_ref, v_ref):
    rh = rh_ref[...]
    rt = rt_ref[...]
    hs = jnp.sum(rh, axis=1, keepdims=True)
    ts = jnp.sum(rt, axis=1, keepdims=True)
    hinv = jnp.where(hs == 0, 0.0, 1.0 / hs)
    tinv = jnp.where(ts == 0, 0.0, 1.0 / ts)
    lr = jnp.dot(rh, le_ref[...], preferred_element_type=jnp.float32) * hinv
    rr = jnp.dot(rt, re_ref[...], preferred_element_type=jnp.float32) * tinv
    u_ref[...] = jax.nn.relu(lr) * al_ref[...]
    v_ref[...] = jax.nn.relu(rr) * ar_ref[...]


def _ku(r_head, r_tail, le, re, al, ar):
    blk = 200
    grid = KG_R // blk
    return pl.pallas_call(
        _ku_body,
        grid=(grid,),
        in_specs=[
            pl.BlockSpec((blk, KG_E), lambda i: (i, 0)),
            pl.BlockSpec((blk, KG_E), lambda i: (i, 0)),
            pl.BlockSpec((KG_E, E_DIM), lambda i: (0, 0)),
            pl.BlockSpec((KG_E, E_DIM), lambda i: (0, 0)),
            pl.BlockSpec((1, E_DIM), lambda i: (0, 0)),
            pl.BlockSpec((1, E_DIM), lambda i: (0, 0)),
        ],
        out_specs=[
            pl.BlockSpec((blk, E_DIM), lambda i: (i, 0)),
            pl.BlockSpec((blk, E_DIM), lambda i: (i, 0)),
        ],
        out_shape=[
            jax.ShapeDtypeStruct((KG_R, E_DIM), jnp.float32),
            jax.ShapeDtypeStruct((KG_R, E_DIM), jnp.float32),
        ],
    )(r_head, r_tail, le, re, al, ar)


def _ks_body(e_ref, u_ref, v_ref, sl_ref, sr_ref):
    e = e_ref[...]
    dn = (((1,), (1,)), ((), ()))
    sl_ref[...] = lax.dot_general(e, u_ref[...], dn,
                                  preferred_element_type=jnp.float32)
    sr_ref[...] = lax.dot_general(e, v_ref[...], dn,
                                  preferred_element_type=jnp.float32)


def _ks(ent, u, v):
    blk = 1000
    grid = KG_E // blk
    return pl.pallas_call(
        _ks_body,
        grid=(grid,),
        in_specs=[
            pl.BlockSpec((blk, E_DIM), lambda i: (i, 0)),
            pl.BlockSpec((KG_R, E_DIM), lambda i: (0, 0)),
            pl.BlockSpec((KG_R, E_DIM), lambda i: (0, 0)),
        ],
        out_specs=[
            pl.BlockSpec((blk, KG_R), lambda i: (i, 0)),
            pl.BlockSpec((blk, KG_R), lambda i: (i, 0)),
        ],
        out_shape=[
            jax.ShapeDtypeStruct((KG_E, KG_R), jnp.float32),
            jax.ShapeDtypeStruct((KG_E, KG_R), jnp.float32),
        ],
    )(ent, u, v)


def _kcomb_body(n_ref, p0_ref, p1_ref, wl_ref, wr_ref,
                g_ref, e_ref, l_ref, r_ref):
    p = p0_ref[...] + p1_ref[...]
    rs = p[:, E_DIM:E_DIM + 1]
    inv = jnp.where(rs == 0, 0.0, 1.0 / rs)
    e_att = jax.nn.relu(p[:, :E_DIM] * inv)
    g = n_ref[...] + BETA1 * e_att
    g_ref[...] = g
    e_ref[...] = jnp.concatenate([g, _ext_cols(g.shape[0])], axis=1)
    l_ref[...] = jnp.dot(g, wl_ref[...], preferred_element_type=jnp.float32)
    r_ref[...] = jnp.dot(g, wr_ref[...], preferred_element_type=jnp.float32)


def _kcomb(name, p0, p1, wl, wr):
    blk = 1000
    grid = KG_E // blk
    return pl.pallas_call(
        _kcomb_body,
        grid=(grid,),
        in_specs=[
            pl.BlockSpec((blk, E_DIM), lambda i: (i, 0)),
            pl.BlockSpec((blk, EXT), lambda i: (i, 0)),
            pl.BlockSpec((blk, EXT), lambda i: (i, 0)),
            pl.BlockSpec((E_DIM, E_DIM), lambda i: (0, 0)),
            pl.BlockSpec((E_DIM, E_DIM), lambda i: (0, 0)),
        ],
        out_specs=[
            pl.BlockSpec((blk, E_DIM), lambda i: (i, 0)),
            pl.BlockSpec((blk, EXT), lambda i: (i, 0)),
            pl.BlockSpec((blk, E_DIM), lambda i: (i, 0)),
            pl.BlockSpec((blk, E_DIM), lambda i: (i, 0)),
        ],
        out_shape=[
            jax.ShapeDtypeStruct((KG_E, E_DIM), jnp.float32),
            jax.ShapeDtypeStruct((KG_E, EXT), jnp.float32),
            jax.ShapeDtypeStruct((KG_E, E_DIM), jnp.float32),
            jax.ShapeDtypeStruct((KG_E, E_DIM), jnp.float32),
        ],
    )(name, p0, p1, wl, wr)


def _kcomb_final_body(n_ref, p0_ref, p1_ref, g_ref):
    p = p0_ref[...] + p1_ref[...]
    rs = p[:, E_DIM:E_DIM + 1]
    inv = jnp.where(rs == 0, 0.0, 1.0 / rs)
    e_att = jax.nn.relu(p[:, :E_DIM] * inv)
    g_ref[...] = n_ref[...] + BETA1 * e_att


def _kcomb_final(name, p0, p1):
    blk = 1000
    grid = KG_E // blk
    return pl.pallas_call(
        _kcomb_final_body,
        grid=(grid,),
        in_specs=[
            pl.BlockSpec((blk, E_DIM), lambda i: (i, 0)),
            pl.BlockSpec((blk, EXT), lambda i: (i, 0)),
            pl.BlockSpec((blk, EXT), lambda i: (i, 0)),
        ],
        out_specs=pl.BlockSpec((blk, E_DIM), lambda i: (i, 0)),
        out_shape=jax.ShapeDtypeStruct((KG_E, E_DIM), jnp.float32),
    )(name, p0, p1)


def _edge_stage(sl, sr, src, dst, rel, ext):
    logit = sl[src, rel] + sr[dst, rel]
    att = jnp.exp(-jnp.where(logit > 0, logit, ALPHA * logit))
    p = jax.ops.segment_sum(att[:, None] * ext[dst], src, num_segments=KG_E)
    return p


def kernel(kg_name_embed, eer_adj_index, eer_adj_data, r_head, r_tail,
           kg_name_w, kg_name_b, w_R_Left, w_R_Right, w_atten_r):
    b2 = kg_name_b.reshape(1, E_DIM)
    al = w_atten_r[:E_DIM, 0].reshape(1, E_DIM)
    ar = w_atten_r[E_DIM:, 0].reshape(1, E_DIM)
    src = eer_adj_index[0]
    dst = eer_adj_index[1]
    rel = eer_adj_data

    name, ext0, l1, r1 = _k1(kg_name_embed, kg_name_w, b2, w_R_Left, w_R_Right)
    u1, v1 = _ku(r_head, r_tail, l1, r1, al, ar)
    sl1, sr1 = _ks(name, u1, v1)
    p1 = _edge_stage(sl1, sr1, src, dst, rel, ext0)
    pz = jnp.zeros_like(p1)
    g1, ext1, l2, r2 = _kcomb(name, p1, pz, w_R_Left, w_R_Right)
    u2, v2 = _ku(r_head, r_tail, l2, r2, al, ar)
    sl2, sr2 = _ks(g1, u2, v2)
    p2 = _edge_stage(sl2, sr2, src, dst, rel, ext1)
    return _kcomb_final(name, p2, pz)


# speedup vs baseline: 7.5814x; 4.4140x over previous
import functools

import jax
import jax.numpy as jnp
from jax import lax
from jax.experimental import pallas as pl
from jax.experimental.pallas import tpu as pltpu
from jax.experimental.pallas import tpu_sc as plsc

KG_E = 10000
KG_R = 1000
E_DIM = 128
N_EDGES = 320000
ALPHA = 0.2
BETA1 = 0.3
EXT = 144


def _ext_cols(n):
    return jnp.where(lax.broadcasted_iota(jnp.int32, (n, 16), 1) == 0, 1.0, 0.0)


def _k1_body(x_ref, w_ref, b_ref, wl_ref, wr_ref, o_ref, e_ref, l_ref, r_ref):
    y = jnp.dot(x_ref[...], w_ref[...], preferred_element_type=jnp.float32)
    y = y + b_ref[...]
    o_ref[...] = y
    e_ref[...] = jnp.concatenate([y, _ext_cols(y.shape[0])], axis=1)
    l_ref[...] = jnp.dot(y, wl_ref[...], preferred_element_type=jnp.float32)
    r_ref[...] = jnp.dot(y, wr_ref[...], preferred_element_type=jnp.float32)


def _k1(kg, w, b2, wl, wr):
    blk = 1000
    grid = KG_E // blk
    return pl.pallas_call(
        _k1_body,
        grid=(grid,),
        in_specs=[
            pl.BlockSpec((blk, 300), lambda i: (i, 0)),
            pl.BlockSpec((300, E_DIM), lambda i: (0, 0)),
            pl.BlockSpec((1, E_DIM), lambda i: (0, 0)),
            pl.BlockSpec((E_DIM, E_DIM), lambda i: (0, 0)),
            pl.BlockSpec((E_DIM, E_DIM), lambda i: (0, 0)),
        ],
        out_specs=[
            pl.BlockSpec((blk, E_DIM), lambda i: (i, 0)),
            pl.BlockSpec((blk, EXT), lambda i: (i, 0)),
            pl.BlockSpec((blk, E_DIM), lambda i: (i, 0)),
            pl.BlockSpec((blk, E_DIM), lambda i: (i, 0)),
        ],
        out_shape=[
            jax.ShapeDtypeStruct((KG_E, E_DIM), jnp.float32),
            jax.ShapeDtypeStruct((KG_E, EXT), jnp.float32),
            jax.ShapeDtypeStruct((KG_E, E_DIM), jnp.float32),
            jax.ShapeDtypeStruct((KG_E, E_DIM), jnp.float32),
        ],
    )(kg, w, b2, wl, wr)


def _ku_body(rh_ref, rt_ref, le_ref, re_ref, al_ref, ar_ref, u_ref, v_ref):
    rh = rh_ref[...]
    rt = rt_ref[...]
    hs = jnp.sum(rh, axis=1, keepdims=True)
    ts = jnp.sum(rt, axis=1, keepdims=True)
    hinv = jnp.where(hs == 0, 0.0, 1.0 / hs)
    tinv = jnp.where(ts == 0, 0.0, 1.0 / ts)
    lr = jnp.dot(rh, le_ref[...], preferred_element_type=jnp.float32) * hinv
    rr = jnp.dot(rt, re_ref[...], preferred_element_type=jnp.float32) * tinv
    u_ref[...] = jax.nn.relu(lr) * al_ref[...]
    v_ref[...] = jax.nn.relu(rr) * ar_ref[...]


def _ku(r_head, r_tail, le, re, al, ar):
    blk = 200
    grid = KG_R // blk
    return pl.pallas_call(
        _ku_body,
        grid=(grid,),
        in_specs=[
            pl.BlockSpec((blk, KG_E), lambda i: (i, 0)),
            pl.BlockSpec((blk, KG_E), lambda i: (i, 0)),
            pl.BlockSpec((KG_E, E_DIM), lambda i: (0, 0)),
            pl.BlockSpec((KG_E, E_DIM), lambda i: (0, 0)),
            pl.BlockSpec((1, E_DIM), lambda i: (0, 0)),
            pl.BlockSpec((1, E_DIM), lambda i: (0, 0)),
        ],
        out_specs=[
            pl.BlockSpec((blk, E_DIM), lambda i: (i, 0)),
            pl.BlockSpec((blk, E_DIM), lambda i: (i, 0)),
        ],
        out_shape=[
            jax.ShapeDtypeStruct((KG_R, E_DIM), jnp.float32),
            jax.ShapeDtypeStruct((KG_R, E_DIM), jnp.float32),
        ],
    )(r_head, r_tail, le, re, al, ar)


def _ks_body(e_ref, u_ref, v_ref, sl_ref, sr_ref):
    e = e_ref[...]
    dn = (((1,), (1,)), ((), ()))
    sl_ref[...] = lax.dot_general(e, u_ref[...], dn,
                                  preferred_element_type=jnp.float32)
    sr_ref[...] = lax.dot_general(e, v_ref[...], dn,
                                  preferred_element_type=jnp.float32)


def _ks(ent, u, v):
    blk = 1000
    grid = KG_E // blk
    return pl.pallas_call(
        _ks_body,
        grid=(grid,),
        in_specs=[
            pl.BlockSpec((blk, E_DIM), lambda i: (i, 0)),
            pl.BlockSpec((KG_R, E_DIM), lambda i: (0, 0)),
            pl.BlockSpec((KG_R, E_DIM), lambda i: (0, 0)),
        ],
        out_specs=[
            pl.BlockSpec((blk, KG_R), lambda i: (i, 0)),
            pl.BlockSpec((blk, KG_R), lambda i: (i, 0)),
        ],
        out_shape=[
            jax.ShapeDtypeStruct((KG_E, KG_R), jnp.float32),
            jax.ShapeDtypeStruct((KG_E, KG_R), jnp.float32),
        ],
    )(ent, u, v)


def _kcomb_body(n_ref, p0_ref, p1_ref, wl_ref, wr_ref,
                g_ref, e_ref, l_ref, r_ref):
    p = p0_ref[...] + p1_ref[...]
    rs = p[:, E_DIM:E_DIM + 1]
    inv = jnp.where(rs == 0, 0.0, 1.0 / rs)
    e_att = jax.nn.relu(p[:, :E_DIM] * inv)
    g = n_ref[...] + BETA1 * e_att
    g_ref[...] = g
    e_ref[...] = jnp.concatenate([g, _ext_cols(g.shape[0])], axis=1)
    l_ref[...] = jnp.dot(g, wl_ref[...], preferred_element_type=jnp.float32)
    r_ref[...] = jnp.dot(g, wr_ref[...], preferred_element_type=jnp.float32)


def _kcomb(name, p0, p1, wl, wr):
    blk = 1000
    grid = KG_E // blk
    return pl.pallas_call(
        _kcomb_body,
        grid=(grid,),
        in_specs=[
            pl.BlockSpec((blk, E_DIM), lambda i: (i, 0)),
            pl.BlockSpec((blk, EXT), lambda i: (i, 0)),
            pl.BlockSpec((blk, EXT), lambda i: (i, 0)),
            pl.BlockSpec((E_DIM, E_DIM), lambda i: (0, 0)),
            pl.BlockSpec((E_DIM, E_DIM), lambda i: (0, 0)),
        ],
        out_specs=[
            pl.BlockSpec((blk, E_DIM), lambda i: (i, 0)),
            pl.BlockSpec((blk, EXT), lambda i: (i, 0)),
            pl.BlockSpec((blk, E_DIM), lambda i: (i, 0)),
            pl.BlockSpec((blk, E_DIM), lambda i: (i, 0)),
        ],
        out_shape=[
            jax.ShapeDtypeStruct((KG_E, E_DIM), jnp.float32),
            jax.ShapeDtypeStruct((KG_E, EXT), jnp.float32),
            jax.ShapeDtypeStruct((KG_E, E_DIM), jnp.float32),
            jax.ShapeDtypeStruct((KG_E, E_DIM), jnp.float32),
        ],
    )(name, p0, p1, wl, wr)


def _kcomb_final_body(n_ref, p0_ref, p1_ref, g_ref):
    p = p0_ref[...] + p1_ref[...]
    rs = p[:, E_DIM:E_DIM + 1]
    inv = jnp.where(rs == 0, 0.0, 1.0 / rs)
    e_att = jax.nn.relu(p[:, :E_DIM] * inv)
    g_ref[...] = n_ref[...] + BETA1 * e_att


def _kcomb_final(name, p0, p1):
    blk = 1000
    grid = KG_E // blk
    return pl.pallas_call(
        _kcomb_final_body,
        grid=(grid,),
        in_specs=[
            pl.BlockSpec((blk, E_DIM), lambda i: (i, 0)),
            pl.BlockSpec((blk, EXT), lambda i: (i, 0)),
            pl.BlockSpec((blk, EXT), lambda i: (i, 0)),
        ],
        out_specs=pl.BlockSpec((blk, E_DIM), lambda i: (i, 0)),
        out_shape=jax.ShapeDtypeStruct((KG_E, E_DIM), jnp.float32),
    )(name, p0, p1)


NC = 2
NS = 16
L = 16
CH = 256
NCHUNK = N_EDGES // CH
IB = CH // 128
_BASE_CH = NCHUNK // (NC * NS)
_EXTRA = NCHUNK - _BASE_CH * NC * NS

_MESH = plsc.VectorSubcoreMesh(core_axis_name="c", subcore_axis_name="s")


def _edge_sc_body(sl_hbm, sr_hbm, src_hbm, dst_hbm, rel_hbm, ext_hbm, zz_hbm,
                  out_hbm,
                  src_v, dst_v, rel_v, idxl_v, idxr_v, slv, srv, att_v,
                  rows_v, acc, sem_i, sem_g, sem_r, sem_w):
    c = lax.axis_index("c")
    s = lax.axis_index("s")
    wid = s * NC + c

    pltpu.sync_copy(zz_hbm.at[pl.ds(0, 624)], acc.at[pl.ds(s * 624, 624)])

    @pl.when(s == 0)
    def _():
        pltpu.sync_copy(zz_hbm.at[pl.ds(624, 16)], acc.at[pl.ds(9984, 16)])

    plsc.subcore_barrier()

    nch = _BASE_CH + jnp.where(wid < _EXTRA, 1, 0)
    ch0 = wid * _BASE_CH + jnp.minimum(wid, _EXTRA)

    def chunk(ci, carry):
        base = (ch0 + ci) * IB
        ld = [pltpu.async_copy(src_hbm.at[pl.ds(base, IB)], src_v, sem_i),
              pltpu.async_copy(dst_hbm.at[pl.ds(base, IB)], dst_v, sem_i),
              pltpu.async_copy(rel_hbm.at[pl.ds(base, IB)], rel_v, sem_i)]
        for cp in ld:
            cp.wait()
        rg = [pltpu.async_copy(ext_hbm.at[dst_v.at[j]],
                               rows_v.at[pl.ds(j * 128, 128)], sem_r)
              for j in range(IB)]
        for j in range(IB):
            for i in range(128 // L):
                sl16 = src_v[j, pl.ds(i * L, L)]
                dl16 = dst_v[j, pl.ds(i * L, L)]
                rl16 = rel_v[j, pl.ds(i * L, L)]
                idxl_v[j, pl.ds(i * L, L)] = sl16 * KG_R + rl16
                idxr_v[j, pl.ds(i * L, L)] = dl16 * KG_R + rl16
        sg = [pltpu.async_copy(sl_hbm.at[idxl_v.at[j]],
                               slv.at[pl.ds(j * 128, 128)], sem_g)
              for j in range(IB)]
        sg += [pltpu.async_copy(sr_hbm.at[idxr_v.at[j]],
                                srv.at[pl.ds(j * 128, 128)], sem_g)
               for j in range(IB)]
        for cp in sg:
            cp.wait()
        for i in range(CH // L):
            x = slv[pl.ds(i * L, L)] + srv[pl.ds(i * L, L)]
            xl = jnp.where(x > 0, x, ALPHA * x)
            att_v[pl.ds(i * L, L)] = jnp.exp(-xl)
        for cp in rg:
            cp.wait()

        def scale_grp(g2, _):
            att16 = att_v[pl.ds(g2 * L, L)]
            for k in range(L):
                r = g2 * L + k
                a = att16[k]
                for jb in range(EXT // L):
                    rows_v[r, pl.ds(jb * L, L)] = rows_v[r, pl.ds(jb * L, L)] * a
            return 0

        lax.fori_loop(0, CH // L, scale_grp, 0)
        sc = [pltpu.async_copy(rows_v.at[pl.ds(j * 128, 128)],
                               acc.at[src_v.at[j]], sem_w, add=True)
              for j in range(IB)]
        for cp in sc:
            cp.wait()
        return 0

    lax.fori_loop(0, nch, chunk, 0)
    plsc.subcore_barrier()

    pltpu.sync_copy(acc.at[pl.ds(s * 624, 624)],
                    out_hbm.at[pl.ds(c * KG_E + s * 624, 624)])

    @pl.when(s == 0)
    def _():
        pltpu.sync_copy(acc.at[pl.ds(9984, 16)],
                        out_hbm.at[pl.ds(c * KG_E + 9984, 16)])


_edge_sc = pl.kernel(
    _edge_sc_body,
    out_type=jax.ShapeDtypeStruct((NC * KG_E, EXT), jnp.float32),
    mesh=_MESH,
    compiler_params=pltpu.CompilerParams(use_tc_tiling_on_sc=False),
    scratch_types=[
        pltpu.VMEM((IB, 128), jnp.int32),
        pltpu.VMEM((IB, 128), jnp.int32),
        pltpu.VMEM((IB, 128), jnp.int32),
        pltpu.VMEM((IB, 128), jnp.int32),
        pltpu.VMEM((IB, 128), jnp.int32),
        pltpu.VMEM((CH,), jnp.float32),
        pltpu.VMEM((CH,), jnp.float32),
        pltpu.VMEM((CH,), jnp.float32),
        pltpu.VMEM((CH, EXT), jnp.float32),
        pltpu.VMEM_SHARED((KG_E, EXT), jnp.float32),
        pltpu.SemaphoreType.DMA,
        pltpu.SemaphoreType.DMA,
        pltpu.SemaphoreType.DMA,
        pltpu.SemaphoreType.DMA,
    ],
)


def _edge_stage(sl, sr, src2, dst2, rel2, ext, zz):
    p = _edge_sc(sl.reshape(-1), sr.reshape(-1), src2, dst2, rel2, ext, zz)
    return p[:KG_E], p[KG_E:]


def kernel(kg_name_embed, eer_adj_index, eer_adj_data, r_head, r_tail,
           kg_name_w, kg_name_b, w_R_Left, w_R_Right, w_atten_r):
    b2 = kg_name_b.reshape(1, E_DIM)
    al = w_atten_r[:E_DIM, 0].reshape(1, E_DIM)
    ar = w_atten_r[E_DIM:, 0].reshape(1, E_DIM)
    src2 = eer_adj_index[0].reshape(-1, 128)
    dst2 = eer_adj_index[1].reshape(-1, 128)
    rel2 = eer_adj_data.reshape(-1, 128)
    zz = jnp.zeros((640, EXT), jnp.float32)

    name, ext0, l1, r1 = _k1(kg_name_embed, kg_name_w, b2, w_R_Left, w_R_Right)
    u1, v1 = _ku(r_head, r_tail, l1, r1, al, ar)
    sl1, sr1 = _ks(name, u1, v1)
    pa1, pb1 = _edge_stage(sl1, sr1, src2, dst2, rel2, ext0, zz)
    g1, ext1, l2, r2 = _kcomb(name, pa1, pb1, w_R_Left, w_R_Right)
    u2, v2 = _ku(r_head, r_tail, l2, r2, al, ar)
    sl2, sr2 = _ks(g1, u2, v2)
    pa2, pb2 = _edge_stage(sl2, sr2, src2, dst2, rel2, ext1, zz)
    return _kcomb_final(name, pa2, pb2)


# speedup vs baseline: 8.2104x; 1.0830x over previous
import functools

import jax
import jax.numpy as jnp
from jax import lax
from jax.experimental import pallas as pl
from jax.experimental.pallas import tpu as pltpu
from jax.experimental.pallas import tpu_sc as plsc

KG_E = 10000
KG_R = 1000
KG_RP = 1024
E_DIM = 128
N_EDGES = 320000
ALPHA = 0.2
BETA1 = 0.3
RHI = KG_RP // 128


def _k1_body(x_ref, w_ref, b_ref, wl_ref, wr_ref, o_ref, l_ref, r_ref):
    y = jnp.dot(x_ref[...], w_ref[...], preferred_element_type=jnp.float32)
    y = y + b_ref[...]
    o_ref[...] = y
    l_ref[...] = jnp.dot(y, wl_ref[...], preferred_element_type=jnp.float32)
    r_ref[...] = jnp.dot(y, wr_ref[...], preferred_element_type=jnp.float32)


def _k1(kg, w, b2, wl, wr):
    blk = 1000
    grid = KG_E // blk
    return pl.pallas_call(
        _k1_body,
        grid=(grid,),
        in_specs=[
            pl.BlockSpec((blk, 300), lambda i: (i, 0)),
            pl.BlockSpec((300, E_DIM), lambda i: (0, 0)),
            pl.BlockSpec((1, E_DIM), lambda i: (0, 0)),
            pl.BlockSpec((E_DIM, E_DIM), lambda i: (0, 0)),
            pl.BlockSpec((E_DIM, E_DIM), lambda i: (0, 0)),
        ],
        out_specs=[
            pl.BlockSpec((blk, E_DIM), lambda i: (i, 0)),
            pl.BlockSpec((blk, E_DIM), lambda i: (i, 0)),
            pl.BlockSpec((blk, E_DIM), lambda i: (i, 0)),
        ],
        out_shape=[
            jax.ShapeDtypeStruct((KG_E, E_DIM), jnp.float32),
            jax.ShapeDtypeStruct((KG_E, E_DIM), jnp.float32),
            jax.ShapeDtypeStruct((KG_E, E_DIM), jnp.float32),
        ],
    )(kg, w, b2, wl, wr)


def _ku_body(rh_ref, rt_ref, le_ref, re_ref, al_ref, ar_ref, u_ref, v_ref):
    rh = rh_ref[...]
    rt = rt_ref[...]
    hs = jnp.sum(rh, axis=1, keepdims=True)
    ts = jnp.sum(rt, axis=1, keepdims=True)
    hinv = jnp.where(hs == 0, 0.0, 1.0 / hs)
    tinv = jnp.where(ts == 0, 0.0, 1.0 / ts)
    lr = jnp.dot(rh, le_ref[...], preferred_element_type=jnp.float32) * hinv
    rr = jnp.dot(rt, re_ref[...], preferred_element_type=jnp.float32) * tinv
    u_ref[...] = jax.nn.relu(lr) * al_ref[...]
    v_ref[...] = jax.nn.relu(rr) * ar_ref[...]


def _ku(r_head, r_tail, le, re, al, ar):
    blk = 200
    grid = KG_R // blk
    return pl.pallas_call(
        _ku_body,
        grid=(grid,),
        in_specs=[
            pl.BlockSpec((blk, KG_E), lambda i: (i, 0)),
            pl.BlockSpec((blk, KG_E), lambda i: (i, 0)),
            pl.BlockSpec((KG_E, E_DIM), lambda i: (0, 0)),
            pl.BlockSpec((KG_E, E_DIM), lambda i: (0, 0)),
            pl.BlockSpec((1, E_DIM), lambda i: (0, 0)),
            pl.BlockSpec((1, E_DIM), lambda i: (0, 0)),
        ],
        out_specs=[
            pl.BlockSpec((blk, E_DIM), lambda i: (i, 0)),
            pl.BlockSpec((blk, E_DIM), lambda i: (i, 0)),
        ],
        out_shape=[
            jax.ShapeDtypeStruct((KG_R, E_DIM), jnp.float32),
            jax.ShapeDtypeStruct((KG_R, E_DIM), jnp.float32),
        ],
    )(r_head, r_tail, le, re, al, ar)


def _ks_body(e_ref, u_ref, v_ref, sl_ref, sr_ref):
    e = e_ref[...]
    dn = (((1,), (1,)), ((), ()))
    sl_ref[...] = lax.dot_general(e, u_ref[...], dn,
                                  preferred_element_type=jnp.float32)[None]
    sr_ref[...] = lax.dot_general(e, v_ref[...], dn,
                                  preferred_element_type=jnp.float32)[None]


def _ks(ent, u_pad, v_pad):
    blk = 1000
    grid = KG_E // blk
    return pl.pallas_call(
        _ks_body,
        grid=(grid, RHI),
        in_specs=[
            pl.BlockSpec((blk, E_DIM), lambda i, j: (i, 0)),
            pl.BlockSpec((128, E_DIM), lambda i, j: (j, 0)),
            pl.BlockSpec((128, E_DIM), lambda i, j: (j, 0)),
        ],
        out_specs=[
            pl.BlockSpec((1, blk, 128), lambda i, j: (j, i, 0)),
            pl.BlockSpec((1, blk, 128), lambda i, j: (j, i, 0)),
        ],
        out_shape=[
            jax.ShapeDtypeStruct((RHI, KG_E, 128), jnp.float32),
            jax.ShapeDtypeStruct((RHI, KG_E, 128), jnp.float32),
        ],
    )(ent, u_pad, v_pad)


def _kcomb_body(n_ref, p0_ref, p1_ref, rs0_ref, rs1_ref, wl_ref, wr_ref,
                g_ref, l_ref, r_ref):
    p = p0_ref[...] + p1_ref[...]
    rs = rs0_ref[...][:, 0:1] + rs1_ref[...][:, 0:1]
    inv = jnp.where(rs == 0, 0.0, 1.0 / rs)
    e_att = jax.nn.relu(p * inv)
    g = n_ref[...] + BETA1 * e_att
    g_ref[...] = g
    l_ref[...] = jnp.dot(g, wl_ref[...], preferred_element_type=jnp.float32)
    r_ref[...] = jnp.dot(g, wr_ref[...], preferred_element_type=jnp.float32)


def _kcomb(name, p0, p1, rs0, rs1, wl, wr):
    blk = 1000
    grid = KG_E // blk
    return pl.pallas_call(
        _kcomb_body,
        grid=(grid,),
        in_specs=[
            pl.BlockSpec((blk, E_DIM), lambda i: (i, 0)),
            pl.BlockSpec((blk, E_DIM), lambda i: (i, 0)),
            pl.BlockSpec((blk, E_DIM), lambda i: (i, 0)),
            pl.BlockSpec((blk, 16), lambda i: (i, 0)),
            pl.BlockSpec((blk, 16), lambda i: (i, 0)),
            pl.BlockSpec((E_DIM, E_DIM), lambda i: (0, 0)),
            pl.BlockSpec((E_DIM, E_DIM), lambda i: (0, 0)),
        ],
        out_specs=[
            pl.BlockSpec((blk, E_DIM), lambda i: (i, 0)),
            pl.BlockSpec((blk, E_DIM), lambda i: (i, 0)),
            pl.BlockSpec((blk, E_DIM), lambda i: (i, 0)),
        ],
        out_shape=[
            jax.ShapeDtypeStruct((KG_E, E_DIM), jnp.float32),
            jax.ShapeDtypeStruct((KG_E, E_DIM), jnp.float32),
            jax.ShapeDtypeStruct((KG_E, E_DIM), jnp.float32),
        ],
    )(name, p0, p1, rs0, rs1, wl, wr)


def _kcomb_final_body(n_ref, p0_ref, p1_ref, rs0_ref, rs1_ref, g_ref):
    p = p0_ref[...] + p1_ref[...]
    rs = rs0_ref[...][:, 0:1] + rs1_ref[...][:, 0:1]
    inv = jnp.where(rs == 0, 0.0, 1.0 / rs)
    e_att = jax.nn.relu(p * inv)
    g_ref[...] = n_ref[...] + BETA1 * e_att


def _kcomb_final(name, p0, p1, rs0, rs1):
    blk = 1000
    grid = KG_E // blk
    return pl.pallas_call(
        _kcomb_final_body,
        grid=(grid,),
        in_specs=[
            pl.BlockSpec((blk, E_DIM), lambda i: (i, 0)),
            pl.BlockSpec((blk, E_DIM), lambda i: (i, 0)),
            pl.BlockSpec((blk, E_DIM), lambda i: (i, 0)),
            pl.BlockSpec((blk, 16), lambda i: (i, 0)),
            pl.BlockSpec((blk, 16), lambda i: (i, 0)),
        ],
        out_specs=pl.BlockSpec((blk, E_DIM), lambda i: (i, 0)),
        out_shape=jax.ShapeDtypeStruct((KG_E, E_DIM), jnp.float32),
    )(name, p0, p1, rs0, rs1)


NC = 2
NS = 16
L = 16
CH = 256
NCHUNK = N_EDGES // CH
IB = CH // 128
_BASE_CH = NCHUNK // (NC * NS)
_EXTRA = NCHUNK - _BASE_CH * NC * NS

_MESH = plsc.VectorSubcoreMesh(core_axis_name="c", subcore_axis_name="s")


def _edge_sc_body(sl_hbm, sr_hbm, src_hbm, dst_hbm, rel_hbm, emb_hbm,
                  zza_hbm, zzb_hbm,
                  out_hbm, ors_hbm,
                  src_v, dst_v, rel_v, idxl_v, idxr_v, slv, srv, att_v,
                  rows_v, rs_rows, acc, acc_rs, sem_i, sem_g, sem_r, sem_w):
    c = lax.axis_index("c")
    s = lax.axis_index("s")
    wid = s * NC + c

    pltpu.sync_copy(zza_hbm.at[pl.ds(0, 624)], acc.at[pl.ds(s * 624, 624)])
    pltpu.sync_copy(zzb_hbm.at[pl.ds(0, 624)], acc_rs.at[pl.ds(s * 624, 624)])

    @pl.when(s == 0)
    def _():
        pltpu.sync_copy(zza_hbm.at[pl.ds(624, 16)], acc.at[pl.ds(9984, 16)])
        pltpu.sync_copy(zzb_hbm.at[pl.ds(624, 16)], acc_rs.at[pl.ds(9984, 16)])

    plsc.subcore_barrier()

    nch = _BASE_CH + jnp.where(wid < _EXTRA, 1, 0)
    ch0 = wid * _BASE_CH + jnp.minimum(wid, _EXTRA)
    lane0 = lax.broadcasted_iota(jnp.int32, (L,), 0)
    m16 = jnp.where(lane0 == 0, 1.0, 0.0).astype(jnp.float32)

    def chunk(ci, carry):
        base = (ch0 + ci) * IB
        ld = [pltpu.async_copy(src_hbm.at[pl.ds(base, IB)], src_v, sem_i),
              pltpu.async_copy(dst_hbm.at[pl.ds(base, IB)], dst_v, sem_i),
              pltpu.async_copy(rel_hbm.at[pl.ds(base, IB)], rel_v, sem_i)]
        for cp in ld:
            cp.wait()
        rg = [pltpu.async_copy(emb_hbm.at[dst_v.at[j]],
                               rows_v.at[pl.ds(j * 128, 128)], sem_r)
              for j in range(IB)]
        for j in range(IB):
            for i in range(128 // L):
                sl16 = src_v[j, pl.ds(i * L, L)]
                dl16 = dst_v[j, pl.ds(i * L, L)]
                rl16 = rel_v[j, pl.ds(i * L, L)]
                rhi = lax.shift_right_logical(rl16, 7) * (KG_E * 128)
                rlo = lax.bitwise_and(rl16, 127) + rhi
                idxl_v[j, pl.ds(i * L, L)] = sl16 * 128 + rlo
                idxr_v[j, pl.ds(i * L, L)] = dl16 * 128 + rlo
        sg = [pltpu.async_copy(sl_hbm.at[idxl_v.at[j]],
                               slv.at[pl.ds(j * 128, 128)], sem_g)
              for j in range(IB)]
        sg += [pltpu.async_copy(sr_hbm.at[idxr_v.at[j]],
                                srv.at[pl.ds(j * 128, 128)], sem_g)
               for j in range(IB)]
        for cp in sg:
            cp.wait()
        for i in range(CH // L):
            x = slv[pl.ds(i * L, L)] + srv[pl.ds(i * L, L)]
            xl = jnp.where(x > 0, x, ALPHA * x)
            att_v[pl.ds(i * L, L)] = jnp.exp(-xl)
        for cp in rg:
            cp.wait()

        def scale_grp(g2, _):
            att16 = att_v[pl.ds(g2 * L, L)]
            for k in range(L):
                r = g2 * L + k
                a = att16[k]
                rs_rows[r, pl.ds(0, L)] = a * m16
                for jb in range(E_DIM // L):
                    rows_v[r, pl.ds(jb * L, L)] = rows_v[r, pl.ds(jb * L, L)] * a
            return 0

        lax.fori_loop(0, CH // L, scale_grp, 0)
        sc = [pltpu.async_copy(rows_v.at[pl.ds(j * 128, 128)],
                               acc.at[src_v.at[j]], sem_w, add=True)
              for j in range(IB)]
        sc += [pltpu.async_copy(rs_rows.at[pl.ds(j * 128, 128)],
                                acc_rs.at[src_v.at[j]], sem_w, add=True)
               for j in range(IB)]
        for cp in sc:
            cp.wait()
        return 0

    lax.fori_loop(0, nch, chunk, 0)
    plsc.subcore_barrier()

    pltpu.sync_copy(acc.at[pl.ds(s * 624, 624)],
                    out_hbm.at[pl.ds(c * KG_E + s * 624, 624)])
    pltpu.sync_copy(acc_rs.at[pl.ds(s * 624, 624)],
                    ors_hbm.at[pl.ds(c * KG_E + s * 624, 624)])

    @pl.when(s == 0)
    def _():
        pltpu.sync_copy(acc.at[pl.ds(9984, 16)],
                        out_hbm.at[pl.ds(c * KG_E + 9984, 16)])
        pltpu.sync_copy(acc_rs.at[pl.ds(9984, 16)],
                        ors_hbm.at[pl.ds(c * KG_E + 9984, 16)])


_edge_sc = pl.kernel(
    _edge_sc_body,
    out_type=[
        jax.ShapeDtypeStruct((NC * KG_E, E_DIM), jnp.float32),
        jax.ShapeDtypeStruct((NC * KG_E, 16), jnp.float32),
    ],
    mesh=_MESH,
    compiler_params=pltpu.CompilerParams(use_tc_tiling_on_sc=False),
    scratch_types=[
        pltpu.VMEM((IB, 128), jnp.int32),
        pltpu.VMEM((IB, 128), jnp.int32),
        pltpu.VMEM((IB, 128), jnp.int32),
        pltpu.VMEM((IB, 128), jnp.int32),
        pltpu.VMEM((IB, 128), jnp.int32),
        pltpu.VMEM((CH,), jnp.float32),
        pltpu.VMEM((CH,), jnp.float32),
        pltpu.VMEM((CH,), jnp.float32),
        pltpu.VMEM((CH, E_DIM), jnp.float32),
        pltpu.VMEM((CH, 16), jnp.float32),
        pltpu.VMEM_SHARED((KG_E, E_DIM), jnp.float32),
        pltpu.VMEM_SHARED((KG_E, 16), jnp.float32),
        pltpu.SemaphoreType.DMA,
        pltpu.SemaphoreType.DMA,
        pltpu.SemaphoreType.DMA,
        pltpu.SemaphoreType.DMA,
    ],
)


def _edge_stage(sl3, sr3, src2, dst2, rel2, emb, zza, zzb):
    p, rs = _edge_sc(sl3.reshape(-1), sr3.reshape(-1), src2, dst2, rel2,
                     emb, zza, zzb)
    return p[:KG_E], p[KG_E:], rs[:KG_E], rs[KG_E:]


def kernel(kg_name_embed, eer_adj_index, eer_adj_data, r_head, r_tail,
           kg_name_w, kg_name_b, w_R_Left, w_R_Right, w_atten_r):
    b2 = kg_name_b.reshape(1, E_DIM)
    al = w_atten_r[:E_DIM, 0].reshape(1, E_DIM)
    ar = w_atten_r[E_DIM:, 0].reshape(1, E_DIM)
    src2 = eer_adj_index[0].reshape(-1, 128)
    dst2 = eer_adj_index[1].reshape(-1, 128)
    rel2 = eer_adj_data.reshape(-1, 128)
    zza = jnp.zeros((640, E_DIM), jnp.float32)
    zzb = jnp.zeros((640, 16), jnp.float32)
    pad = ((0, KG_RP - KG_R), (0, 0))

    name, l1, r1 = _k1(kg_name_embed, kg_name_w, b2, w_R_Left, w_R_Right)
    u1, v1 = _ku(r_head, r_tail, l1, r1, al, ar)
    sl1, sr1 = _ks(name, jnp.pad(u1, pad), jnp.pad(v1, pad))
    pa1, pb1, ra1, rb1 = _edge_stage(sl1, sr1, src2, dst2, rel2, name,
                                     zza, zzb)
    g1, l2, r2 = _kcomb(name, pa1, pb1, ra1, rb1, w_R_Left, w_R_Right)
    u2, v2 = _ku(r_head, r_tail, l2, r2, al, ar)
    sl2, sr2 = _ks(g1, jnp.pad(u2, pad), jnp.pad(v2, pad))
    pa2, pb2, ra2, rb2 = _edge_stage(sl2, sr2, src2, dst2, rel2, g1,
                                     zza, zzb)
    return _kcomb_final(name, pa2, pb2, ra2, rb2)


# speedup vs baseline: 8.7367x; 1.0641x over previous
import functools

import jax
import jax.numpy as jnp
from jax import lax
from jax.experimental import pallas as pl
from jax.experimental.pallas import tpu as pltpu
from jax.experimental.pallas import tpu_sc as plsc

KG_E = 10000
KG_R = 1000
KG_RP = 1024
E_DIM = 128
N_EDGES = 320000
ALPHA = 0.2
BETA1 = 0.3
RHI = KG_RP // 128


def _k1_body(x_ref, w_ref, b_ref, wl_ref, wr_ref, o_ref, l_ref, r_ref):
    y = jnp.dot(x_ref[...], w_ref[...], preferred_element_type=jnp.float32)
    y = y + b_ref[...]
    o_ref[...] = y
    l_ref[...] = jnp.dot(y, wl_ref[...], preferred_element_type=jnp.float32)
    r_ref[...] = jnp.dot(y, wr_ref[...], preferred_element_type=jnp.float32)


def _k1(kg, w, b2, wl, wr):
    blk = 1000
    grid = KG_E // blk
    return pl.pallas_call(
        _k1_body,
        grid=(grid,),
        in_specs=[
            pl.BlockSpec((blk, 300), lambda i: (i, 0)),
            pl.BlockSpec((300, E_DIM), lambda i: (0, 0)),
            pl.BlockSpec((1, E_DIM), lambda i: (0, 0)),
            pl.BlockSpec((E_DIM, E_DIM), lambda i: (0, 0)),
            pl.BlockSpec((E_DIM, E_DIM), lambda i: (0, 0)),
        ],
        out_specs=[
            pl.BlockSpec((blk, E_DIM), lambda i: (i, 0)),
            pl.BlockSpec((blk, E_DIM), lambda i: (i, 0)),
            pl.BlockSpec((blk, E_DIM), lambda i: (i, 0)),
        ],
        out_shape=[
            jax.ShapeDtypeStruct((KG_E, E_DIM), jnp.float32),
            jax.ShapeDtypeStruct((KG_E, E_DIM), jnp.float32),
            jax.ShapeDtypeStruct((KG_E, E_DIM), jnp.float32),
        ],
    )(kg, w, b2, wl, wr)


def _ku_body(rh_ref, rt_ref, le_ref, re_ref, al_ref, ar_ref, u_ref, v_ref):
    rh = rh_ref[...]
    rt = rt_ref[...]
    hs = jnp.sum(rh, axis=1, keepdims=True)
    ts = jnp.sum(rt, axis=1, keepdims=True)
    hinv = jnp.where(hs == 0, 0.0, 1.0 / hs)
    tinv = jnp.where(ts == 0, 0.0, 1.0 / ts)
    lr = jnp.dot(rh, le_ref[...], preferred_element_type=jnp.float32) * hinv
    rr = jnp.dot(rt, re_ref[...], preferred_element_type=jnp.float32) * tinv
    u_ref[...] = jax.nn.relu(lr) * al_ref[...]
    v_ref[...] = jax.nn.relu(rr) * ar_ref[...]


def _ku(r_head, r_tail, le, re, al, ar):
    blk = 200
    grid = KG_R // blk
    return pl.pallas_call(
        _ku_body,
        grid=(grid,),
        in_specs=[
            pl.BlockSpec((blk, KG_E), lambda i: (i, 0)),
            pl.BlockSpec((blk, KG_E), lambda i: (i, 0)),
            pl.BlockSpec((KG_E, E_DIM), lambda i: (0, 0)),
            pl.BlockSpec((KG_E, E_DIM), lambda i: (0, 0)),
            pl.BlockSpec((1, E_DIM), lambda i: (0, 0)),
            pl.BlockSpec((1, E_DIM), lambda i: (0, 0)),
        ],
        out_specs=[
            pl.BlockSpec((blk, E_DIM), lambda i: (i, 0)),
            pl.BlockSpec((blk, E_DIM), lambda i: (i, 0)),
        ],
        out_shape=[
            jax.ShapeDtypeStruct((KG_R, E_DIM), jnp.float32),
            jax.ShapeDtypeStruct((KG_R, E_DIM), jnp.float32),
        ],
    )(r_head, r_tail, le, re, al, ar)


def _ks_body(e_ref, u_ref, v_ref, sl_ref, sr_ref):
    e = e_ref[...]
    dn = (((1,), (1,)), ((), ()))
    sl_ref[...] = lax.dot_general(e, u_ref[...], dn,
                                  preferred_element_type=jnp.float32)[None]
    sr_ref[...] = lax.dot_general(e, v_ref[...], dn,
                                  preferred_element_type=jnp.float32)[None]


def _ks(ent, u_pad, v_pad):
    blk = 1000
    grid = KG_E // blk
    return pl.pallas_call(
        _ks_body,
        grid=(grid, RHI),
        in_specs=[
            pl.BlockSpec((blk, E_DIM), lambda i, j: (i, 0)),
            pl.BlockSpec((128, E_DIM), lambda i, j: (j, 0)),
            pl.BlockSpec((128, E_DIM), lambda i, j: (j, 0)),
        ],
        out_specs=[
            pl.BlockSpec((1, blk, 128), lambda i, j: (j, i, 0)),
            pl.BlockSpec((1, blk, 128), lambda i, j: (j, i, 0)),
        ],
        out_shape=[
            jax.ShapeDtypeStruct((RHI, KG_E, 128), jnp.float32),
            jax.ShapeDtypeStruct((RHI, KG_E, 128), jnp.float32),
        ],
    )(ent, u_pad, v_pad)


def _kcomb_body(n_ref, p0_ref, p1_ref, rs0_ref, rs1_ref, wl_ref, wr_ref,
                g_ref, l_ref, r_ref):
    p = p0_ref[...] + p1_ref[...]
    rs = rs0_ref[...][:, 0:1] + rs1_ref[...][:, 0:1]
    inv = jnp.where(rs == 0, 0.0, 1.0 / rs)
    e_att = jax.nn.relu(p * inv)
    g = n_ref[...] + BETA1 * e_att
    g_ref[...] = g
    l_ref[...] = jnp.dot(g, wl_ref[...], preferred_element_type=jnp.float32)
    r_ref[...] = jnp.dot(g, wr_ref[...], preferred_element_type=jnp.float32)


def _kcomb(name, p0, p1, rs0, rs1, wl, wr):
    blk = 1000
    grid = KG_E // blk
    return pl.pallas_call(
        _kcomb_body,
        grid=(grid,),
        in_specs=[
            pl.BlockSpec((blk, E_DIM), lambda i: (i, 0)),
            pl.BlockSpec((blk, E_DIM), lambda i: (i, 0)),
            pl.BlockSpec((blk, E_DIM), lambda i: (i, 0)),
            pl.BlockSpec((blk, 16), lambda i: (i, 0)),
            pl.BlockSpec((blk, 16), lambda i: (i, 0)),
            pl.BlockSpec((E_DIM, E_DIM), lambda i: (0, 0)),
            pl.BlockSpec((E_DIM, E_DIM), lambda i: (0, 0)),
        ],
        out_specs=[
            pl.BlockSpec((blk, E_DIM), lambda i: (i, 0)),
            pl.BlockSpec((blk, E_DIM), lambda i: (i, 0)),
            pl.BlockSpec((blk, E_DIM), lambda i: (i, 0)),
        ],
        out_shape=[
            jax.ShapeDtypeStruct((KG_E, E_DIM), jnp.float32),
            jax.ShapeDtypeStruct((KG_E, E_DIM), jnp.float32),
            jax.ShapeDtypeStruct((KG_E, E_DIM), jnp.float32),
        ],
    )(name, p0, p1, rs0, rs1, wl, wr)


def _kcomb_final_body(n_ref, p0_ref, p1_ref, rs0_ref, rs1_ref, g_ref):
    p = p0_ref[...] + p1_ref[...]
    rs = rs0_ref[...][:, 0:1] + rs1_ref[...][:, 0:1]
    inv = jnp.where(rs == 0, 0.0, 1.0 / rs)
    e_att = jax.nn.relu(p * inv)
    g_ref[...] = n_ref[...] + BETA1 * e_att


def _kcomb_final(name, p0, p1, rs0, rs1):
    blk = 1000
    grid = KG_E // blk
    return pl.pallas_call(
        _kcomb_final_body,
        grid=(grid,),
        in_specs=[
            pl.BlockSpec((blk, E_DIM), lambda i: (i, 0)),
            pl.BlockSpec((blk, E_DIM), lambda i: (i, 0)),
            pl.BlockSpec((blk, E_DIM), lambda i: (i, 0)),
            pl.BlockSpec((blk, 16), lambda i: (i, 0)),
            pl.BlockSpec((blk, 16), lambda i: (i, 0)),
        ],
        out_specs=pl.BlockSpec((blk, E_DIM), lambda i: (i, 0)),
        out_shape=jax.ShapeDtypeStruct((KG_E, E_DIM), jnp.float32),
    )(name, p0, p1, rs0, rs1)


NC = 2
NS = 16
L = 16
CH = 128
NCHUNK = N_EDGES // CH
_BASE_CH = NCHUNK // (NC * NS)
_EXTRA = NCHUNK - _BASE_CH * NC * NS

_MESH = plsc.VectorSubcoreMesh(core_axis_name="c", subcore_axis_name="s")


def _edge_sc_body(sl_hbm, sr_hbm, src_hbm, dst_hbm, rel_hbm, emb_hbm,
                  zza_hbm, zzb_hbm,
                  out_hbm, ors_hbm,
                  *scr):
    (src0, dst0, rel0, il0, ir0, slv0, srv0, att0, rw0, rsr0,
     src1, dst1, rel1, il1, ir1, slv1, srv1, att1, rw1, rsr1,
     acc, acc_rs,
     si0, sg0, sro0, sw0, si1, sg1, sro1, sw1) = scr
    bufs = ((src0, dst0, rel0, il0, ir0, slv0, srv0, att0, rw0, rsr0,
             si0, sg0, sro0, sw0),
            (src1, dst1, rel1, il1, ir1, slv1, srv1, att1, rw1, rsr1,
             si1, sg1, sro1, sw1))
    c = lax.axis_index("c")
    s = lax.axis_index("s")
    wid = s * NC + c

    pltpu.sync_copy(zza_hbm.at[pl.ds(0, 624)], acc.at[pl.ds(s * 624, 624)])
    pltpu.sync_copy(zzb_hbm.at[pl.ds(0, 624)], acc_rs.at[pl.ds(s * 624, 624)])

    @pl.when(s == 0)
    def _():
        pltpu.sync_copy(zza_hbm.at[pl.ds(624, 16)], acc.at[pl.ds(9984, 16)])
        pltpu.sync_copy(zzb_hbm.at[pl.ds(624, 16)], acc_rs.at[pl.ds(9984, 16)])

    plsc.subcore_barrier()

    nch = _BASE_CH + jnp.where(wid < _EXTRA, 1, 0)
    ch0 = wid * _BASE_CH + jnp.minimum(wid, _EXTRA)
    lane0 = lax.broadcasted_iota(jnp.int32, (L,), 0)
    m16 = jnp.where(lane0 == 0, 1.0, 0.0).astype(jnp.float32)

    def fire_ld(ci, b):
        srcv, dstv, relv = b[0], b[1], b[2]
        base = ch0 + ci
        return [pltpu.async_copy(src_hbm.at[pl.ds(base, 1)], srcv, b[10]),
                pltpu.async_copy(dst_hbm.at[pl.ds(base, 1)], dstv, b[10]),
                pltpu.async_copy(rel_hbm.at[pl.ds(base, 1)], relv, b[10])]

    def fire_gathers(b):
        srcv, dstv, relv, ilv, irv = b[0], b[1], b[2], b[3], b[4]
        for i in range(CH // L):
            sl16 = srcv[0, pl.ds(i * L, L)]
            dl16 = dstv[0, pl.ds(i * L, L)]
            rl16 = relv[0, pl.ds(i * L, L)]
            rhi = lax.shift_right_logical(rl16, 7) * (KG_E * 128)
            rlo = lax.bitwise_and(rl16, 127) + rhi
            ilv[0, pl.ds(i * L, L)] = sl16 * 128 + rlo
            irv[0, pl.ds(i * L, L)] = dl16 * 128 + rlo
        rg = pltpu.async_copy(emb_hbm.at[dstv.at[0]], b[8], b[12])
        g1 = pltpu.async_copy(sl_hbm.at[ilv.at[0]], b[5], b[11])
        g2 = pltpu.async_copy(sr_hbm.at[irv.at[0]], b[6], b[11])
        return rg, g1, g2

    def att_and_scale(b):
        slvv, srvv, attv, rowsv, rsv = b[5], b[6], b[7], b[8], b[9]
        for i in range(CH // L):
            x = slvv[pl.ds(i * L, L)] + srvv[pl.ds(i * L, L)]
            xl = jnp.where(x > 0, x, ALPHA * x)
            attv[pl.ds(i * L, L)] = jnp.exp(-xl)

    def scale(b):
        attv, rowsv, rsv = b[7], b[8], b[9]

        def scale_grp(g2, _):
            att16 = attv[pl.ds(g2 * L, L)]
            for k in range(L):
                r = g2 * L + k
                a = att16[k]
                rsv[r, pl.ds(0, L)] = a * m16
                for jb in range(E_DIM // L):
                    rowsv[r, pl.ds(jb * L, L)] = rowsv[r, pl.ds(jb * L, L)] * a
            return 0

        lax.fori_loop(0, CH // L, scale_grp, 0)

    def fire_scatter(b):
        srcv, rowsv, rsv = b[0], b[8], b[9]
        return [pltpu.async_copy(rowsv, acc.at[srcv.at[0]], b[13], add=True),
                pltpu.async_copy(rsv, acc_rs.at[srcv.at[0]], b[13], add=True)]

    def pair(t, carry):
        ciA = t * 2
        ciB = ciA + 1
        A, B = bufs
        ldA = fire_ld(ciA, A)
        ldB = fire_ld(ciB, B)
        for cp in ldA:
            cp.wait()
        rgA, gA1, gA2 = fire_gathers(A)
        for cp in ldB:
            cp.wait()
        rgB, gB1, gB2 = fire_gathers(B)
        gA1.wait()
        gA2.wait()
        att_and_scale(A)
        rgA.wait()
        scale(A)
        scA = fire_scatter(A)
        gB1.wait()
        gB2.wait()
        att_and_scale(B)
        rgB.wait()
        scale(B)
        scB = fire_scatter(B)
        for cp in scA:
            cp.wait()
        for cp in scB:
            cp.wait()
        return 0

    lax.fori_loop(0, _BASE_CH // 2, pair, 0)

    @pl.when(wid < _EXTRA)
    def _():
        A = bufs[0]
        ld = fire_ld(_BASE_CH, A)
        for cp in ld:
            cp.wait()
        rgA, gA1, gA2 = fire_gathers(A)
        gA1.wait()
        gA2.wait()
        att_and_scale(A)
        rgA.wait()
        scale(A)
        for cp in fire_scatter(A):
            cp.wait()

    plsc.subcore_barrier()

    pltpu.sync_copy(acc.at[pl.ds(s * 624, 624)],
                    out_hbm.at[pl.ds(c * KG_E + s * 624, 624)])
    pltpu.sync_copy(acc_rs.at[pl.ds(s * 624, 624)],
                    ors_hbm.at[pl.ds(c * KG_E + s * 624, 624)])

    @pl.when(s == 0)
    def _():
        pltpu.sync_copy(acc.at[pl.ds(9984, 16)],
                        out_hbm.at[pl.ds(c * KG_E + 9984, 16)])
        pltpu.sync_copy(acc_rs.at[pl.ds(9984, 16)],
                        ors_hbm.at[pl.ds(c * KG_E + 9984, 16)])


_edge_sc = pl.kernel(
    _edge_sc_body,
    out_type=[
        jax.ShapeDtypeStruct((NC * KG_E, E_DIM), jnp.float32),
        jax.ShapeDtypeStruct((NC * KG_E, 16), jnp.float32),
    ],
    mesh=_MESH,
    compiler_params=pltpu.CompilerParams(use_tc_tiling_on_sc=False),
    scratch_types=(
        2 * [
            pltpu.VMEM((1, 128), jnp.int32),
            pltpu.VMEM((1, 128), jnp.int32),
            pltpu.VMEM((1, 128), jnp.int32),
            pltpu.VMEM((1, 128), jnp.int32),
            pltpu.VMEM((1, 128), jnp.int32),
            pltpu.VMEM((CH,), jnp.float32),
            pltpu.VMEM((CH,), jnp.float32),
            pltpu.VMEM((CH,), jnp.float32),
            pltpu.VMEM((CH, E_DIM), jnp.float32),
            pltpu.VMEM((CH, 16), jnp.float32),
        ]
        + [
            pltpu.VMEM_SHARED((KG_E, E_DIM), jnp.float32),
            pltpu.VMEM_SHARED((KG_E, 16), jnp.float32),
        ]
        + 8 * [pltpu.SemaphoreType.DMA]
    ),
)


def _edge_stage(sl3, sr3, src2, dst2, rel2, emb, zza, zzb):
    p, rs = _edge_sc(sl3.reshape(-1), sr3.reshape(-1), src2, dst2, rel2,
                     emb, zza, zzb)
    return p[:KG_E], p[KG_E:], rs[:KG_E], rs[KG_E:]


def kernel(kg_name_embed, eer_adj_index, eer_adj_data, r_head, r_tail,
           kg_name_w, kg_name_b, w_R_Left, w_R_Right, w_atten_r):
    b2 = kg_name_b.reshape(1, E_DIM)
    al = w_atten_r[:E_DIM, 0].reshape(1, E_DIM)
    ar = w_atten_r[E_DIM:, 0].reshape(1, E_DIM)
    src2 = eer_adj_index[0].reshape(-1, 128)
    dst2 = eer_adj_index[1].reshape(-1, 128)
    rel2 = eer_adj_data.reshape(-1, 128)
    zza = jnp.zeros((640, E_DIM), jnp.float32)
    zzb = jnp.zeros((640, 16), jnp.float32)
    pad = ((0, KG_RP - KG_R), (0, 0))

    name, l1, r1 = _k1(kg_name_embed, kg_name_w, b2, w_R_Left, w_R_Right)
    u1, v1 = _ku(r_head, r_tail, l1, r1, al, ar)
    sl1, sr1 = _ks(name, jnp.pad(u1, pad), jnp.pad(v1, pad))
    pa1, pb1, ra1, rb1 = _edge_stage(sl1, sr1, src2, dst2, rel2, name,
                                     zza, zzb)
    g1, l2, r2 = _kcomb(name, pa1, pb1, ra1, rb1, w_R_Left, w_R_Right)
    u2, v2 = _ku(r_head, r_tail, l2, r2, al, ar)
    sl2, sr2 = _ks(g1, jnp.pad(u2, pad), jnp.pad(v2, pad))
    pa2, pb2, ra2, rb2 = _edge_stage(sl2, sr2, src2, dst2, rel2, g1,
                                     zza, zzb)
    return _kcomb_final(name, pa2, pb2, ra2, rb2)


# speedup vs baseline: 9.0409x; 1.0348x over previous
import functools

import jax
import jax.numpy as jnp
from jax import lax
from jax.experimental import pallas as pl
from jax.experimental.pallas import tpu as pltpu
from jax.experimental.pallas import tpu_sc as plsc

KG_E = 10000
KG_R = 1000
KG_RP = 1024
E_DIM = 128
N_EDGES = 320000
ALPHA = 0.2
BETA1 = 0.3
RHI = KG_RP // 128


def _k1_body(x_ref, w_ref, b_ref, wl_ref, wr_ref, o_ref, l_ref, r_ref):
    y = jnp.dot(x_ref[...], w_ref[...], preferred_element_type=jnp.float32)
    y = y + b_ref[...]
    o_ref[...] = y
    l_ref[...] = jnp.dot(y, wl_ref[...], preferred_element_type=jnp.float32)
    r_ref[...] = jnp.dot(y, wr_ref[...], preferred_element_type=jnp.float32)


def _k1(kg, w, b2, wl, wr):
    blk = 1000
    grid = KG_E // blk
    return pl.pallas_call(
        _k1_body,
        grid=(grid,),
        in_specs=[
            pl.BlockSpec((blk, 300), lambda i: (i, 0)),
            pl.BlockSpec((300, E_DIM), lambda i: (0, 0)),
            pl.BlockSpec((1, E_DIM), lambda i: (0, 0)),
            pl.BlockSpec((E_DIM, E_DIM), lambda i: (0, 0)),
            pl.BlockSpec((E_DIM, E_DIM), lambda i: (0, 0)),
        ],
        out_specs=[
            pl.BlockSpec((blk, E_DIM), lambda i: (i, 0)),
            pl.BlockSpec((blk, E_DIM), lambda i: (i, 0)),
            pl.BlockSpec((blk, E_DIM), lambda i: (i, 0)),
        ],
        out_shape=[
            jax.ShapeDtypeStruct((KG_E, E_DIM), jnp.float32),
            jax.ShapeDtypeStruct((KG_E, E_DIM), jnp.float32),
            jax.ShapeDtypeStruct((KG_E, E_DIM), jnp.float32),
        ],
    )(kg, w, b2, wl, wr)


def _ku_body(rh_ref, rt_ref, le_ref, re_ref, al_ref, ar_ref, u_ref, v_ref):
    rh = rh_ref[...]
    rt = rt_ref[...]
    hs = jnp.sum(rh, axis=1, keepdims=True)
    ts = jnp.sum(rt, axis=1, keepdims=True)
    hinv = jnp.where(hs == 0, 0.0, 1.0 / hs)
    tinv = jnp.where(ts == 0, 0.0, 1.0 / ts)
    lr = jnp.dot(rh, le_ref[...], preferred_element_type=jnp.float32) * hinv
    rr = jnp.dot(rt, re_ref[...], preferred_element_type=jnp.float32) * tinv
    u_ref[...] = jax.nn.relu(lr) * al_ref[...]
    v_ref[...] = jax.nn.relu(rr) * ar_ref[...]


def _ku(r_head, r_tail, le, re, al, ar):
    blk = 200
    grid = KG_R // blk
    return pl.pallas_call(
        _ku_body,
        grid=(grid,),
        in_specs=[
            pl.BlockSpec((blk, KG_E), lambda i: (i, 0)),
            pl.BlockSpec((blk, KG_E), lambda i: (i, 0)),
            pl.BlockSpec((KG_E, E_DIM), lambda i: (0, 0)),
            pl.BlockSpec((KG_E, E_DIM), lambda i: (0, 0)),
            pl.BlockSpec((1, E_DIM), lambda i: (0, 0)),
            pl.BlockSpec((1, E_DIM), lambda i: (0, 0)),
        ],
        out_specs=[
            pl.BlockSpec((blk, E_DIM), lambda i: (i, 0)),
            pl.BlockSpec((blk, E_DIM), lambda i: (i, 0)),
        ],
        out_shape=[
            jax.ShapeDtypeStruct((KG_R, E_DIM), jnp.float32),
            jax.ShapeDtypeStruct((KG_R, E_DIM), jnp.float32),
        ],
    )(r_head, r_tail, le, re, al, ar)


def _ks_body(e_ref, u_ref, v_ref, sl_ref, sr_ref):
    e = e_ref[...].astype(jnp.bfloat16)
    dn = (((1,), (1,)), ((), ()))
    sl_ref[...] = lax.dot_general(e, u_ref[...].astype(jnp.bfloat16), dn,
                                  preferred_element_type=jnp.float32)[None]
    sr_ref[...] = lax.dot_general(e, v_ref[...].astype(jnp.bfloat16), dn,
                                  preferred_element_type=jnp.float32)[None]


def _ks(ent, u_pad, v_pad):
    blk = 1000
    grid = KG_E // blk
    return pl.pallas_call(
        _ks_body,
        grid=(grid, RHI),
        in_specs=[
            pl.BlockSpec((blk, E_DIM), lambda i, j: (i, 0)),
            pl.BlockSpec((128, E_DIM), lambda i, j: (j, 0)),
            pl.BlockSpec((128, E_DIM), lambda i, j: (j, 0)),
        ],
        out_specs=[
            pl.BlockSpec((1, blk, 128), lambda i, j: (j, i, 0)),
            pl.BlockSpec((1, blk, 128), lambda i, j: (j, i, 0)),
        ],
        out_shape=[
            jax.ShapeDtypeStruct((RHI, KG_E, 128), jnp.float32),
            jax.ShapeDtypeStruct((RHI, KG_E, 128), jnp.float32),
        ],
    )(ent, u_pad, v_pad)


def _kcomb_body(n_ref, p0_ref, p1_ref, rs0_ref, rs1_ref, wl_ref, wr_ref,
                g_ref, l_ref, r_ref):
    p = p0_ref[...] + p1_ref[...]
    rs = rs0_ref[...][:, 0:1] + rs1_ref[...][:, 0:1]
    inv = jnp.where(rs == 0, 0.0, 1.0 / rs)
    e_att = jax.nn.relu(p * inv)
    g = n_ref[...] + BETA1 * e_att
    g_ref[...] = g
    l_ref[...] = jnp.dot(g, wl_ref[...], preferred_element_type=jnp.float32)
    r_ref[...] = jnp.dot(g, wr_ref[...], preferred_element_type=jnp.float32)


def _kcomb(name, p, rs, wl, wr):
    blk = 1000
    grid = KG_E // blk
    nb = grid
    return pl.pallas_call(
        _kcomb_body,
        grid=(grid,),
        in_specs=[
            pl.BlockSpec((blk, E_DIM), lambda i: (i, 0)),
            pl.BlockSpec((blk, E_DIM), lambda i: (i, 0)),
            pl.BlockSpec((blk, E_DIM), lambda i: (i + nb, 0)),
            pl.BlockSpec((blk, 16), lambda i: (i, 0)),
            pl.BlockSpec((blk, 16), lambda i: (i + nb, 0)),
            pl.BlockSpec((E_DIM, E_DIM), lambda i: (0, 0)),
            pl.BlockSpec((E_DIM, E_DIM), lambda i: (0, 0)),
        ],
        out_specs=[
            pl.BlockSpec((blk, E_DIM), lambda i: (i, 0)),
            pl.BlockSpec((blk, E_DIM), lambda i: (i, 0)),
            pl.BlockSpec((blk, E_DIM), lambda i: (i, 0)),
        ],
        out_shape=[
            jax.ShapeDtypeStruct((KG_E, E_DIM), jnp.float32),
            jax.ShapeDtypeStruct((KG_E, E_DIM), jnp.float32),
            jax.ShapeDtypeStruct((KG_E, E_DIM), jnp.float32),
        ],
    )(name, p, p, rs, rs, wl, wr)


def _kcomb_final_body(n_ref, p0_ref, p1_ref, rs0_ref, rs1_ref, g_ref):
    p = p0_ref[...] + p1_ref[...]
    rs = rs0_ref[...][:, 0:1] + rs1_ref[...][:, 0:1]
    inv = jnp.where(rs == 0, 0.0, 1.0 / rs)
    e_att = jax.nn.relu(p * inv)
    g_ref[...] = n_ref[...] + BETA1 * e_att


def _kcomb_final(name, p, rs):
    blk = 1000
    grid = KG_E // blk
    nb = grid
    return pl.pallas_call(
        _kcomb_final_body,
        grid=(grid,),
        in_specs=[
            pl.BlockSpec((blk, E_DIM), lambda i: (i, 0)),
            pl.BlockSpec((blk, E_DIM), lambda i: (i, 0)),
            pl.BlockSpec((blk, E_DIM), lambda i: (i + nb, 0)),
            pl.BlockSpec((blk, 16), lambda i: (i, 0)),
            pl.BlockSpec((blk, 16), lambda i: (i + nb, 0)),
        ],
        out_specs=pl.BlockSpec((blk, E_DIM), lambda i: (i, 0)),
        out_shape=jax.ShapeDtypeStruct((KG_E, E_DIM), jnp.float32),
    )(name, p, p, rs, rs)


NC = 2
NS = 16
L = 16
CH = 128
NCHUNK = N_EDGES // CH
_BASE_CH = NCHUNK // (NC * NS)
_EXTRA = NCHUNK - _BASE_CH * NC * NS

_MESH = plsc.VectorSubcoreMesh(core_axis_name="c", subcore_axis_name="s")


def _edge_sc_body(sl_hbm, sr_hbm, src_hbm, dst_hbm, rel_hbm, emb_hbm,
                  zza_hbm, zzb_hbm,
                  out_hbm, ors_hbm,
                  *scr):
    (src0, dst0, rel0, il0, ir0, slv0, srv0, att0, rw0, rsr0,
     src1, dst1, rel1, il1, ir1, slv1, srv1, att1, rw1, rsr1,
     acc, acc_rs,
     si0, sg0, sro0, sw0, si1, sg1, sro1, sw1) = scr
    bufs = ((src0, dst0, rel0, il0, ir0, slv0, srv0, att0, rw0, rsr0,
             si0, sg0, sro0, sw0),
            (src1, dst1, rel1, il1, ir1, slv1, srv1, att1, rw1, rsr1,
             si1, sg1, sro1, sw1))
    c = lax.axis_index("c")
    s = lax.axis_index("s")
    wid = s * NC + c

    pltpu.sync_copy(zza_hbm.at[pl.ds(0, 624)], acc.at[pl.ds(s * 624, 624)])
    pltpu.sync_copy(zzb_hbm.at[pl.ds(0, 624)], acc_rs.at[pl.ds(s * 624, 624)])

    @pl.when(s == 0)
    def _():
        pltpu.sync_copy(zza_hbm.at[pl.ds(624, 16)], acc.at[pl.ds(9984, 16)])
        pltpu.sync_copy(zzb_hbm.at[pl.ds(624, 16)], acc_rs.at[pl.ds(9984, 16)])

    plsc.subcore_barrier()

    nch = _BASE_CH + jnp.where(wid < _EXTRA, 1, 0)
    ch0 = wid * _BASE_CH + jnp.minimum(wid, _EXTRA)
    lane0 = lax.broadcasted_iota(jnp.int32, (L,), 0)
    m16 = jnp.where(lane0 == 0, 1.0, 0.0).astype(jnp.float32)

    def fire_ld(ci, b):
        srcv, dstv, relv = b[0], b[1], b[2]
        base = ch0 + ci
        return [pltpu.async_copy(src_hbm.at[pl.ds(base, 1)], srcv, b[10]),
                pltpu.async_copy(dst_hbm.at[pl.ds(base, 1)], dstv, b[10]),
                pltpu.async_copy(rel_hbm.at[pl.ds(base, 1)], relv, b[10])]

    def fire_gathers(b):
        srcv, dstv, relv, ilv, irv = b[0], b[1], b[2], b[3], b[4]
        for i in range(CH // L):
            sl16 = srcv[0, pl.ds(i * L, L)]
            dl16 = dstv[0, pl.ds(i * L, L)]
            rl16 = relv[0, pl.ds(i * L, L)]
            rhi = lax.shift_right_logical(rl16, 7) * (KG_E * 128)
            rlo = lax.bitwise_and(rl16, 127) + rhi
            ilv[0, pl.ds(i * L, L)] = sl16 * 128 + rlo
            irv[0, pl.ds(i * L, L)] = dl16 * 128 + rlo
        rg = pltpu.async_copy(emb_hbm.at[dstv.at[0]], b[8], b[12])
        g1 = pltpu.async_copy(sl_hbm.at[ilv.at[0]], b[5], b[11])
        g2 = pltpu.async_copy(sr_hbm.at[irv.at[0]], b[6], b[11])
        return rg, g1, g2

    def att_and_scale(b):
        slvv, srvv, attv, rowsv, rsv = b[5], b[6], b[7], b[8], b[9]
        for i in range(CH // L):
            x = slvv[pl.ds(i * L, L)] + srvv[pl.ds(i * L, L)]
            xl = jnp.where(x > 0, x, ALPHA * x)
            attv[pl.ds(i * L, L)] = jnp.exp(-xl)

    def scale(b):
        attv, rowsv, rsv = b[7], b[8], b[9]

        def scale_grp(g2, _):
            att16 = attv[pl.ds(g2 * L, L)]
            for k in range(L):
                r = g2 * L + k
                a = att16[k]
                rsv[r, pl.ds(0, L)] = a * m16
                for jb in range(E_DIM // L):
                    rowsv[r, pl.ds(jb * L, L)] = rowsv[r, pl.ds(jb * L, L)] * a
            return 0

        lax.fori_loop(0, CH // L, scale_grp, 0)

    def fire_scatter(b):
        srcv, rowsv, rsv = b[0], b[8], b[9]
        return [pltpu.async_copy(rowsv, acc.at[srcv.at[0]], b[13], add=True),
                pltpu.async_copy(rsv, acc_rs.at[srcv.at[0]], b[13], add=True)]

    def pair(t, carry):
        ciA = t * 2
        ciB = ciA + 1
        A, B = bufs
        ldA = fire_ld(ciA, A)
        ldB = fire_ld(ciB, B)
        for cp in ldA:
            cp.wait()
        rgA, gA1, gA2 = fire_gathers(A)
        for cp in ldB:
            cp.wait()
        rgB, gB1, gB2 = fire_gathers(B)
        gA1.wait()
        gA2.wait()
        att_and_scale(A)
        rgA.wait()
        scale(A)
        scA = fire_scatter(A)
        gB1.wait()
        gB2.wait()
        att_and_scale(B)
        rgB.wait()
        scale(B)
        scB = fire_scatter(B)
        for cp in scA:
            cp.wait()
        for cp in scB:
            cp.wait()
        return 0

    lax.fori_loop(0, _BASE_CH // 2, pair, 0)

    @pl.when(wid < _EXTRA)
    def _():
        A = bufs[0]
        ld = fire_ld(_BASE_CH, A)
        for cp in ld:
            cp.wait()
        rgA, gA1, gA2 = fire_gathers(A)
        gA1.wait()
        gA2.wait()
        att_and_scale(A)
        rgA.wait()
        scale(A)
        for cp in fire_scatter(A):
            cp.wait()

    plsc.subcore_barrier()

    pltpu.sync_copy(acc.at[pl.ds(s * 624, 624)],
                    out_hbm.at[pl.ds(c * KG_E + s * 624, 624)])
    pltpu.sync_copy(acc_rs.at[pl.ds(s * 624, 624)],
                    ors_hbm.at[pl.ds(c * KG_E + s * 624, 624)])

    @pl.when(s == 0)
    def _():
        pltpu.sync_copy(acc.at[pl.ds(9984, 16)],
                        out_hbm.at[pl.ds(c * KG_E + 9984, 16)])
        pltpu.sync_copy(acc_rs.at[pl.ds(9984, 16)],
                        ors_hbm.at[pl.ds(c * KG_E + 9984, 16)])


_edge_sc = pl.kernel(
    _edge_sc_body,
    out_type=[
        jax.ShapeDtypeStruct((NC * KG_E, E_DIM), jnp.float32),
        jax.ShapeDtypeStruct((NC * KG_E, 16), jnp.float32),
    ],
    mesh=_MESH,
    compiler_params=pltpu.CompilerParams(use_tc_tiling_on_sc=False),
    scratch_types=(
        2 * [
            pltpu.VMEM((1, 128), jnp.int32),
            pltpu.VMEM((1, 128), jnp.int32),
            pltpu.VMEM((1, 128), jnp.int32),
            pltpu.VMEM((1, 128), jnp.int32),
            pltpu.VMEM((1, 128), jnp.int32),
            pltpu.VMEM((CH,), jnp.float32),
            pltpu.VMEM((CH,), jnp.float32),
            pltpu.VMEM((CH,), jnp.float32),
            pltpu.VMEM((CH, E_DIM), jnp.float32),
            pltpu.VMEM((CH, 16), jnp.float32),
        ]
        + [
            pltpu.VMEM_SHARED((KG_E, E_DIM), jnp.float32),
            pltpu.VMEM_SHARED((KG_E, 16), jnp.float32),
        ]
        + 8 * [pltpu.SemaphoreType.DMA]
    ),
)


def _edge_stage(sl3, sr3, src2, dst2, rel2, emb, zza, zzb):
    return _edge_sc(sl3.reshape(-1), sr3.reshape(-1), src2, dst2, rel2,
                    emb, zza, zzb)


def kernel(kg_name_embed, eer_adj_index, eer_adj_data, r_head, r_tail,
           kg_name_w, kg_name_b, w_R_Left, w_R_Right, w_atten_r):
    b2 = kg_name_b.reshape(1, E_DIM)
    al = w_atten_r[:E_DIM, 0].reshape(1, E_DIM)
    ar = w_atten_r[E_DIM:, 0].reshape(1, E_DIM)
    src2 = eer_adj_index[0].reshape(-1, 128)
    dst2 = eer_adj_index[1].reshape(-1, 128)
    rel2 = eer_adj_data.reshape(-1, 128)
    zza = jnp.zeros((640, E_DIM), jnp.float32)
    zzb = jnp.zeros((640, 16), jnp.float32)
    pad = ((0, KG_RP - KG_R), (0, 0))

    name, l1, r1 = _k1(kg_name_embed, kg_name_w, b2, w_R_Left, w_R_Right)
    u1, v1 = _ku(r_head, r_tail, l1, r1, al, ar)
    sl1, sr1 = _ks(name, jnp.pad(u1, pad), jnp.pad(v1, pad))
    p1, rs1x = _edge_stage(sl1, sr1, src2, dst2, rel2, name, zza, zzb)
    g1, l2, r2 = _kcomb(name, p1, rs1x, w_R_Left, w_R_Right)
    u2, v2 = _ku(r_head, r_tail, l2, r2, al, ar)
    sl2, sr2 = _ks(g1, jnp.pad(u2, pad), jnp.pad(v2, pad))
    p2, rs2x = _edge_stage(sl2, sr2, src2, dst2, rel2, g1, zza, zzb)
    return _kcomb_final(name, p2, rs2x)


# speedup vs baseline: 10.6234x; 1.1750x over previous
import functools

import jax
import jax.numpy as jnp
from jax import lax
from jax.experimental import pallas as pl
from jax.experimental.pallas import tpu as pltpu
from jax.experimental.pallas import tpu_sc as plsc

KG_E = 10000
KG_R = 1000
KG_RP = 1024
E_DIM = 128
N_EDGES = 320000
ALPHA = 0.2
BETA1 = 0.3
RHI = KG_RP // 128


def _k1_body(x_ref, w_ref, b_ref, wl_ref, wr_ref, o_ref, l_ref, r_ref):
    y = jnp.dot(x_ref[...], w_ref[...], preferred_element_type=jnp.float32)
    y = y + b_ref[...]
    o_ref[...] = y
    l_ref[...] = jnp.dot(y, wl_ref[...], preferred_element_type=jnp.float32)
    r_ref[...] = jnp.dot(y, wr_ref[...], preferred_element_type=jnp.float32)


def _k1(kg, w, b2, wl, wr):
    blk = 1000
    grid = KG_E // blk
    return pl.pallas_call(
        _k1_body,
        grid=(grid,),
        in_specs=[
            pl.BlockSpec((blk, 300), lambda i: (i, 0)),
            pl.BlockSpec((300, E_DIM), lambda i: (0, 0)),
            pl.BlockSpec((1, E_DIM), lambda i: (0, 0)),
            pl.BlockSpec((E_DIM, E_DIM), lambda i: (0, 0)),
            pl.BlockSpec((E_DIM, E_DIM), lambda i: (0, 0)),
        ],
        out_specs=[
            pl.BlockSpec((blk, E_DIM), lambda i: (i, 0)),
            pl.BlockSpec((blk, E_DIM), lambda i: (i, 0)),
            pl.BlockSpec((blk, E_DIM), lambda i: (i, 0)),
        ],
        out_shape=[
            jax.ShapeDtypeStruct((KG_E, E_DIM), jnp.float32),
            jax.ShapeDtypeStruct((KG_E, E_DIM), jnp.float32),
            jax.ShapeDtypeStruct((KG_E, E_DIM), jnp.float32),
        ],
    )(kg, w, b2, wl, wr)


def _ku_body(rh_ref, rt_ref, le_ref, re_ref, al_ref, ar_ref, u_ref, v_ref):
    rh = rh_ref[...]
    rt = rt_ref[...]
    hs = jnp.sum(rh, axis=1, keepdims=True)
    ts = jnp.sum(rt, axis=1, keepdims=True)
    hinv = jnp.where(hs == 0, 0.0, 1.0 / hs)
    tinv = jnp.where(ts == 0, 0.0, 1.0 / ts)
    lr = jnp.dot(rh, le_ref[...], preferred_element_type=jnp.float32) * hinv
    rr = jnp.dot(rt, re_ref[...], preferred_element_type=jnp.float32) * tinv
    u_ref[...] = jax.nn.relu(lr) * al_ref[...]
    v_ref[...] = jax.nn.relu(rr) * ar_ref[...]


def _ku(r_head, r_tail, le, re, al, ar):
    blk = 200
    grid = KG_R // blk
    return pl.pallas_call(
        _ku_body,
        grid=(grid,),
        in_specs=[
            pl.BlockSpec((blk, KG_E), lambda i: (i, 0)),
            pl.BlockSpec((blk, KG_E), lambda i: (i, 0)),
            pl.BlockSpec((KG_E, E_DIM), lambda i: (0, 0)),
            pl.BlockSpec((KG_E, E_DIM), lambda i: (0, 0)),
            pl.BlockSpec((1, E_DIM), lambda i: (0, 0)),
            pl.BlockSpec((1, E_DIM), lambda i: (0, 0)),
        ],
        out_specs=[
            pl.BlockSpec((blk, E_DIM), lambda i: (i, 0)),
            pl.BlockSpec((blk, E_DIM), lambda i: (i, 0)),
        ],
        out_shape=[
            jax.ShapeDtypeStruct((KG_R, E_DIM), jnp.float32),
            jax.ShapeDtypeStruct((KG_R, E_DIM), jnp.float32),
        ],
    )(r_head, r_tail, le, re, al, ar)


def _ks_body(e_ref, u_ref, v_ref, sl_ref, sr_ref):
    e = e_ref[...].astype(jnp.bfloat16)
    dn = (((1,), (1,)), ((), ()))
    sl_ref[...] = lax.dot_general(e, u_ref[...].astype(jnp.bfloat16), dn,
                                  preferred_element_type=jnp.float32)[None]
    sr_ref[...] = lax.dot_general(e, v_ref[...].astype(jnp.bfloat16), dn,
                                  preferred_element_type=jnp.float32)[None]


def _ks(ent, u_pad, v_pad):
    blk = 2000
    grid = KG_E // blk
    return pl.pallas_call(
        _ks_body,
        grid=(grid, RHI),
        in_specs=[
            pl.BlockSpec((blk, E_DIM), lambda i, j: (i, 0)),
            pl.BlockSpec((128, E_DIM), lambda i, j: (j, 0)),
            pl.BlockSpec((128, E_DIM), lambda i, j: (j, 0)),
        ],
        out_specs=[
            pl.BlockSpec((1, blk, 128), lambda i, j: (j, i, 0)),
            pl.BlockSpec((1, blk, 128), lambda i, j: (j, i, 0)),
        ],
        out_shape=[
            jax.ShapeDtypeStruct((RHI, KG_E, 128), jnp.float32),
            jax.ShapeDtypeStruct((RHI, KG_E, 128), jnp.float32),
        ],
    )(ent, u_pad, v_pad)


def _kcomb_body(n_ref, p0_ref, p1_ref, rs0_ref, rs1_ref, wl_ref, wr_ref,
                g_ref, l_ref, r_ref):
    p = p0_ref[...] + p1_ref[...]
    rs = rs0_ref[...] + rs1_ref[...]
    inv = jnp.where(rs == 0, 0.0, 1.0 / rs)
    e_att = jax.nn.relu(p * inv)
    g = n_ref[...] + BETA1 * e_att
    g_ref[...] = g
    l_ref[...] = jnp.dot(g, wl_ref[...], preferred_element_type=jnp.float32)
    r_ref[...] = jnp.dot(g, wr_ref[...], preferred_element_type=jnp.float32)


def _kcomb(name, p, rs, wl, wr):
    blk = 1000
    grid = KG_E // blk
    nb = grid
    return pl.pallas_call(
        _kcomb_body,
        grid=(grid,),
        in_specs=[
            pl.BlockSpec((blk, E_DIM), lambda i: (i, 0)),
            pl.BlockSpec((blk, E_DIM), lambda i: (i, 0)),
            pl.BlockSpec((blk, E_DIM), lambda i: (i + nb, 0)),
            pl.BlockSpec((blk, 1), lambda i: (i, 0)),
            pl.BlockSpec((blk, 1), lambda i: (i + nb, 0)),
            pl.BlockSpec((E_DIM, E_DIM), lambda i: (0, 0)),
            pl.BlockSpec((E_DIM, E_DIM), lambda i: (0, 0)),
        ],
        out_specs=[
            pl.BlockSpec((blk, E_DIM), lambda i: (i, 0)),
            pl.BlockSpec((blk, E_DIM), lambda i: (i, 0)),
            pl.BlockSpec((blk, E_DIM), lambda i: (i, 0)),
        ],
        out_shape=[
            jax.ShapeDtypeStruct((KG_E, E_DIM), jnp.float32),
            jax.ShapeDtypeStruct((KG_E, E_DIM), jnp.float32),
            jax.ShapeDtypeStruct((KG_E, E_DIM), jnp.float32),
        ],
    )(name, p, p, rs, rs, wl, wr)


def _kcomb_final_body(n_ref, p0_ref, p1_ref, rs0_ref, rs1_ref, g_ref):
    p = p0_ref[...] + p1_ref[...]
    rs = rs0_ref[...] + rs1_ref[...]
    inv = jnp.where(rs == 0, 0.0, 1.0 / rs)
    e_att = jax.nn.relu(p * inv)
    g_ref[...] = n_ref[...] + BETA1 * e_att


def _kcomb_final(name, p, rs):
    blk = 1000
    grid = KG_E // blk
    nb = grid
    return pl.pallas_call(
        _kcomb_final_body,
        grid=(grid,),
        in_specs=[
            pl.BlockSpec((blk, E_DIM), lambda i: (i, 0)),
            pl.BlockSpec((blk, E_DIM), lambda i: (i, 0)),
            pl.BlockSpec((blk, E_DIM), lambda i: (i + nb, 0)),
            pl.BlockSpec((blk, 1), lambda i: (i, 0)),
            pl.BlockSpec((blk, 1), lambda i: (i + nb, 0)),
        ],
        out_specs=pl.BlockSpec((blk, E_DIM), lambda i: (i, 0)),
        out_shape=jax.ShapeDtypeStruct((KG_E, E_DIM), jnp.float32),
    )(name, p, p, rs, rs)


NC = 2
NS = 16
L = 16
CH = 128
NCHUNK = N_EDGES // CH
_BASE_CH = NCHUNK // (NC * NS)
_EXTRA = NCHUNK - _BASE_CH * NC * NS

_MESH = plsc.VectorSubcoreMesh(core_axis_name="c", subcore_axis_name="s")


def _edge_sc_body(sl_hbm, sr_hbm, src_hbm, dst_hbm, rel_hbm, emb_hbm,
                  zza_hbm, zzb_hbm,
                  out_hbm, ors_hbm,
                  *scr):
    (src0, dst0, rel0, il0, ir0, slv0, srv0, att0, rw0,
     src1, dst1, rel1, il1, ir1, slv1, srv1, att1, rw1,
     acc, acc_rs,
     si0, sg0, sro0, sw0, si1, sg1, sro1, sw1) = scr
    bufs = ((src0, dst0, rel0, il0, ir0, slv0, srv0, att0, rw0,
             si0, sg0, sro0, sw0),
            (src1, dst1, rel1, il1, ir1, slv1, srv1, att1, rw1,
             si1, sg1, sro1, sw1))
    c = lax.axis_index("c")
    s = lax.axis_index("s")
    wid = s * NC + c

    pltpu.sync_copy(zza_hbm.at[pl.ds(0, 624)], acc.at[pl.ds(s * 624, 624)])
    pltpu.sync_copy(zzb_hbm.at[pl.ds(0, 624)], acc_rs.at[pl.ds(s * 624, 624)])

    @pl.when(s == 0)
    def _():
        pltpu.sync_copy(zza_hbm.at[pl.ds(624, 16)], acc.at[pl.ds(9984, 16)])
        pltpu.sync_copy(zzb_hbm.at[pl.ds(624, 16)], acc_rs.at[pl.ds(9984, 16)])

    plsc.subcore_barrier()

    ch0 = wid * _BASE_CH + jnp.minimum(wid, _EXTRA)

    def fire_ld(ci, b):
        srcv, dstv, relv = b[0], b[1], b[2]
        base = ch0 + ci
        return [pltpu.async_copy(src_hbm.at[pl.ds(base, 1)], srcv, b[9]),
                pltpu.async_copy(dst_hbm.at[pl.ds(base, 1)], dstv, b[9]),
                pltpu.async_copy(rel_hbm.at[pl.ds(base, 1)], relv, b[9])]

    def fire_gathers(b):
        srcv, dstv, relv, ilv, irv = b[0], b[1], b[2], b[3], b[4]
        for i in range(CH // L):
            sl16 = srcv[0, pl.ds(i * L, L)]
            dl16 = dstv[0, pl.ds(i * L, L)]
            rl16 = relv[0, pl.ds(i * L, L)]
            rhi = lax.shift_right_logical(rl16, 7) * (KG_E * 128)
            rlo = lax.bitwise_and(rl16, 127) + rhi
            ilv[0, pl.ds(i * L, L)] = sl16 * 128 + rlo
            irv[0, pl.ds(i * L, L)] = dl16 * 128 + rlo
        rg = pltpu.async_copy(emb_hbm.at[dstv.at[0]], b[8], b[11])
        g1 = pltpu.async_copy(sl_hbm.at[ilv.at[0]], b[5], b[10])
        g2 = pltpu.async_copy(sr_hbm.at[irv.at[0]], b[6], b[10])
        return rg, g1, g2

    def att_and_scale(b):
        slvv, srvv, attv = b[5], b[6], b[7]
        for i in range(CH // L):
            x = slvv[pl.ds(i * L, L)] + srvv[pl.ds(i * L, L)]
            xl = jnp.where(x > 0, x, ALPHA * x)
            attv[pl.ds(i * L, L)] = jnp.exp(-xl)

    def scale(b):
        attv, rowsv = b[7], b[8]

        def scale_grp(g2, _):
            att16 = attv[pl.ds(g2 * L, L)]
            for k in range(L):
                r = g2 * L + k
                a = att16[k]
                for jb in range(E_DIM // L):
                    rowsv[r, pl.ds(jb * L, L)] = rowsv[r, pl.ds(jb * L, L)] * a
            return 0

        lax.fori_loop(0, CH // L, scale_grp, 0)

    def fire_scatter(b):
        srcv, attv, rowsv = b[0], b[7], b[8]
        return [pltpu.async_copy(rowsv, acc.at[srcv.at[0]], b[12], add=True),
                pltpu.async_copy(attv, acc_rs.at[srcv.at[0]], b[12], add=True)]

    def pair(t, carry):
        ciA = t * 2
        ciB = ciA + 1
        A, B = bufs
        ldA = fire_ld(ciA, A)
        ldB = fire_ld(ciB, B)
        for cp in ldA:
            cp.wait()
        rgA, gA1, gA2 = fire_gathers(A)
        for cp in ldB:
            cp.wait()
        rgB, gB1, gB2 = fire_gathers(B)
        gA1.wait()
        gA2.wait()
        att_and_scale(A)
        rgA.wait()
        scale(A)
        scA = fire_scatter(A)
        gB1.wait()
        gB2.wait()
        att_and_scale(B)
        rgB.wait()
        scale(B)
        scB = fire_scatter(B)
        for cp in scA:
            cp.wait()
        for cp in scB:
            cp.wait()
        return 0

    lax.fori_loop(0, _BASE_CH // 2, pair, 0)

    @pl.when(wid < _EXTRA)
    def _():
        A = bufs[0]
        ld = fire_ld(_BASE_CH, A)
        for cp in ld:
            cp.wait()
        rgA, gA1, gA2 = fire_gathers(A)
        gA1.wait()
        gA2.wait()
        att_and_scale(A)
        rgA.wait()
        scale(A)
        for cp in fire_scatter(A):
            cp.wait()

    plsc.subcore_barrier()

    pltpu.sync_copy(acc.at[pl.ds(s * 624, 624)],
                    out_hbm.at[pl.ds(c * KG_E + s * 624, 624)])
    pltpu.sync_copy(acc_rs.at[pl.ds(s * 624, 624)],
                    ors_hbm.at[pl.ds(c * KG_E + s * 624, 624)])

    @pl.when(s == 0)
    def _():
        pltpu.sync_copy(acc.at[pl.ds(9984, 16)],
                        out_hbm.at[pl.ds(c * KG_E + 9984, 16)])
        pltpu.sync_copy(acc_rs.at[pl.ds(9984, 16)],
                        ors_hbm.at[pl.ds(c * KG_E + 9984, 16)])


_edge_sc = pl.kernel(
    _edge_sc_body,
    out_type=[
        jax.ShapeDtypeStruct((NC * KG_E, E_DIM), jnp.float32),
        jax.ShapeDtypeStruct((NC * KG_E,), jnp.float32),
    ],
    mesh=_MESH,
    compiler_params=pltpu.CompilerParams(use_tc_tiling_on_sc=False),
    scratch_types=(
        2 * [
            pltpu.VMEM((1, 128), jnp.int32),
            pltpu.VMEM((1, 128), jnp.int32),
            pltpu.VMEM((1, 128), jnp.int32),
            pltpu.VMEM((1, 128), jnp.int32),
            pltpu.VMEM((1, 128), jnp.int32),
            pltpu.VMEM((CH,), jnp.float32),
            pltpu.VMEM((CH,), jnp.float32),
            pltpu.VMEM((CH,), jnp.float32),
            pltpu.VMEM((CH, E_DIM), jnp.float32),
        ]
        + [
            pltpu.VMEM_SHARED((KG_E, E_DIM), jnp.float32),
            pltpu.VMEM_SHARED((KG_E,), jnp.float32),
        ]
        + 8 * [pltpu.SemaphoreType.DMA]
    ),
)


def _edge_stage(sl3, sr3, src2, dst2, rel2, emb, zza, zzb):
    p, rs = _edge_sc(sl3.reshape(-1), sr3.reshape(-1), src2, dst2, rel2,
                     emb, zza, zzb)
    return p, rs.reshape(NC * KG_E, 1)


def kernel(kg_name_embed, eer_adj_index, eer_adj_data, r_head, r_tail,
           kg_name_w, kg_name_b, w_R_Left, w_R_Right, w_atten_r):
    b2 = kg_name_b.reshape(1, E_DIM)
    al = w_atten_r[:E_DIM, 0].reshape(1, E_DIM)
    ar = w_atten_r[E_DIM:, 0].reshape(1, E_DIM)
    src2 = eer_adj_index[0].reshape(-1, 128)
    dst2 = eer_adj_index[1].reshape(-1, 128)
    rel2 = eer_adj_data.reshape(-1, 128)
    zza = jnp.zeros((640, E_DIM), jnp.float32)
    zzb = jnp.zeros((640,), jnp.float32)
    pad = ((0, KG_RP - KG_R), (0, 0))

    name, l1, r1 = _k1(kg_name_embed, kg_name_w, b2, w_R_Left, w_R_Right)
    u1, v1 = _ku(r_head, r_tail, l1, r1, al, ar)
    sl1, sr1 = _ks(name, jnp.pad(u1, pad), jnp.pad(v1, pad))
    p1, rs1x = _edge_stage(sl1, sr1, src2, dst2, rel2, name, zza, zzb)
    g1, l2, r2 = _kcomb(name, p1, rs1x, w_R_Left, w_R_Right)
    u2, v2 = _ku(r_head, r_tail, l2, r2, al, ar)
    sl2, sr2 = _ks(g1, jnp.pad(u2, pad), jnp.pad(v2, pad))
    p2, rs2x = _edge_stage(sl2, sr2, src2, dst2, rel2, g1, zza, zzb)
    return _kcomb_final(name, p2, rs2x)


# speedup vs baseline: 10.6424x; 1.0018x over previous
import functools

import jax
import jax.numpy as jnp
from jax import lax
from jax.experimental import pallas as pl
from jax.experimental.pallas import tpu as pltpu
from jax.experimental.pallas import tpu_sc as plsc

KG_E = 10000
KG_R = 1000
KG_RP = 1024
E_DIM = 128
N_EDGES = 320000
ALPHA = 0.2
BETA1 = 0.3
RHI = KG_RP // 128


def _k1_body(x_ref, w_ref, b_ref, wl_ref, wr_ref, o_ref, l_ref, r_ref):
    y = jnp.dot(x_ref[...], w_ref[...], preferred_element_type=jnp.float32)
    y = y + b_ref[...]
    o_ref[...] = y
    l_ref[...] = jnp.dot(y, wl_ref[...], preferred_element_type=jnp.float32)
    r_ref[...] = jnp.dot(y, wr_ref[...], preferred_element_type=jnp.float32)


def _k1(kg, w, b2, wl, wr):
    blk = 1000
    grid = KG_E // blk
    return pl.pallas_call(
        _k1_body,
        grid=(grid,),
        in_specs=[
            pl.BlockSpec((blk, 300), lambda i: (i, 0)),
            pl.BlockSpec((300, E_DIM), lambda i: (0, 0)),
            pl.BlockSpec((1, E_DIM), lambda i: (0, 0)),
            pl.BlockSpec((E_DIM, E_DIM), lambda i: (0, 0)),
            pl.BlockSpec((E_DIM, E_DIM), lambda i: (0, 0)),
        ],
        out_specs=[
            pl.BlockSpec((blk, E_DIM), lambda i: (i, 0)),
            pl.BlockSpec((blk, E_DIM), lambda i: (i, 0)),
            pl.BlockSpec((blk, E_DIM), lambda i: (i, 0)),
        ],
        out_shape=[
            jax.ShapeDtypeStruct((KG_E, E_DIM), jnp.float32),
            jax.ShapeDtypeStruct((KG_E, E_DIM), jnp.float32),
            jax.ShapeDtypeStruct((KG_E, E_DIM), jnp.float32),
        ],
    )(kg, w, b2, wl, wr)


def _ku_body(rh_ref, rt_ref, le_ref, re_ref, al_ref, ar_ref, u_ref, v_ref):
    rh = rh_ref[...]
    rt = rt_ref[...]
    hs = jnp.sum(rh, axis=1, keepdims=True)
    ts = jnp.sum(rt, axis=1, keepdims=True)
    hinv = jnp.where(hs == 0, 0.0, 1.0 / hs)
    tinv = jnp.where(ts == 0, 0.0, 1.0 / ts)
    lr = jnp.dot(rh, le_ref[...], preferred_element_type=jnp.float32) * hinv
    rr = jnp.dot(rt, re_ref[...], preferred_element_type=jnp.float32) * tinv
    u_ref[...] = jax.nn.relu(lr) * al_ref[...]
    v_ref[...] = jax.nn.relu(rr) * ar_ref[...]


def _ku(r_head, r_tail, le, re, al, ar):
    blk = 200
    grid = KG_R // blk
    return pl.pallas_call(
        _ku_body,
        grid=(grid,),
        in_specs=[
            pl.BlockSpec((blk, KG_E), lambda i: (i, 0)),
            pl.BlockSpec((blk, KG_E), lambda i: (i, 0)),
            pl.BlockSpec((KG_E, E_DIM), lambda i: (0, 0)),
            pl.BlockSpec((KG_E, E_DIM), lambda i: (0, 0)),
            pl.BlockSpec((1, E_DIM), lambda i: (0, 0)),
            pl.BlockSpec((1, E_DIM), lambda i: (0, 0)),
        ],
        out_specs=[
            pl.BlockSpec((blk, E_DIM), lambda i: (i, 0)),
            pl.BlockSpec((blk, E_DIM), lambda i: (i, 0)),
        ],
        out_shape=[
            jax.ShapeDtypeStruct((KG_R, E_DIM), jnp.float32),
            jax.ShapeDtypeStruct((KG_R, E_DIM), jnp.float32),
        ],
    )(r_head, r_tail, le, re, al, ar)


def _ks_body(e_ref, u_ref, v_ref, sl_ref, sr_ref):
    e = e_ref[...].astype(jnp.bfloat16)
    dn = (((1,), (1,)), ((), ()))
    sl_ref[...] = lax.dot_general(e, u_ref[...].astype(jnp.bfloat16), dn,
                                  preferred_element_type=jnp.float32)[None]
    sr_ref[...] = lax.dot_general(e, v_ref[...].astype(jnp.bfloat16), dn,
                                  preferred_element_type=jnp.float32)[None]


def _ks(ent, u_pad, v_pad):
    blk = 2000
    grid = KG_E // blk
    return pl.pallas_call(
        _ks_body,
        grid=(grid, RHI),
        in_specs=[
            pl.BlockSpec((blk, E_DIM), lambda i, j: (i, 0)),
            pl.BlockSpec((128, E_DIM), lambda i, j: (j, 0)),
            pl.BlockSpec((128, E_DIM), lambda i, j: (j, 0)),
        ],
        out_specs=[
            pl.BlockSpec((1, blk, 128), lambda i, j: (j, i, 0)),
            pl.BlockSpec((1, blk, 128), lambda i, j: (j, i, 0)),
        ],
        out_shape=[
            jax.ShapeDtypeStruct((RHI, KG_E, 128), jnp.float32),
            jax.ShapeDtypeStruct((RHI, KG_E, 128), jnp.float32),
        ],
    )(ent, u_pad, v_pad)


def _kcomb_body(n_ref, p0_ref, p1_ref, rs0_ref, rs1_ref, wl_ref, wr_ref,
                g_ref, l_ref, r_ref):
    p = p0_ref[...] + p1_ref[...]
    rs = rs0_ref[...] + rs1_ref[...]
    inv = jnp.where(rs == 0, 0.0, 1.0 / rs)
    e_att = jax.nn.relu(p * inv)
    g = n_ref[...] + BETA1 * e_att
    g_ref[...] = g
    l_ref[...] = jnp.dot(g, wl_ref[...], preferred_element_type=jnp.float32)
    r_ref[...] = jnp.dot(g, wr_ref[...], preferred_element_type=jnp.float32)


def _kcomb(name, p, rs, wl, wr):
    blk = 1000
    grid = KG_E // blk
    nb = grid
    return pl.pallas_call(
        _kcomb_body,
        grid=(grid,),
        in_specs=[
            pl.BlockSpec((blk, E_DIM), lambda i: (i, 0)),
            pl.BlockSpec((blk, E_DIM), lambda i: (i, 0)),
            pl.BlockSpec((blk, E_DIM), lambda i: (i + nb, 0)),
            pl.BlockSpec((blk, 1), lambda i: (i, 0)),
            pl.BlockSpec((blk, 1), lambda i: (i + nb, 0)),
            pl.BlockSpec((E_DIM, E_DIM), lambda i: (0, 0)),
            pl.BlockSpec((E_DIM, E_DIM), lambda i: (0, 0)),
        ],
        out_specs=[
            pl.BlockSpec((blk, E_DIM), lambda i: (i, 0)),
            pl.BlockSpec((blk, E_DIM), lambda i: (i, 0)),
            pl.BlockSpec((blk, E_DIM), lambda i: (i, 0)),
        ],
        out_shape=[
            jax.ShapeDtypeStruct((KG_E, E_DIM), jnp.float32),
            jax.ShapeDtypeStruct((KG_E, E_DIM), jnp.float32),
            jax.ShapeDtypeStruct((KG_E, E_DIM), jnp.float32),
        ],
    )(name, p, p, rs, rs, wl, wr)


def _kcomb_final_body(n_ref, p0_ref, p1_ref, rs0_ref, rs1_ref, g_ref):
    p = p0_ref[...] + p1_ref[...]
    rs = rs0_ref[...] + rs1_ref[...]
    inv = jnp.where(rs == 0, 0.0, 1.0 / rs)
    e_att = jax.nn.relu(p * inv)
    g_ref[...] = n_ref[...] + BETA1 * e_att


def _kcomb_final(name, p, rs):
    blk = 1000
    grid = KG_E // blk
    nb = grid
    return pl.pallas_call(
        _kcomb_final_body,
        grid=(grid,),
        in_specs=[
            pl.BlockSpec((blk, E_DIM), lambda i: (i, 0)),
            pl.BlockSpec((blk, E_DIM), lambda i: (i, 0)),
            pl.BlockSpec((blk, E_DIM), lambda i: (i + nb, 0)),
            pl.BlockSpec((blk, 1), lambda i: (i, 0)),
            pl.BlockSpec((blk, 1), lambda i: (i + nb, 0)),
        ],
        out_specs=pl.BlockSpec((blk, E_DIM), lambda i: (i, 0)),
        out_shape=jax.ShapeDtypeStruct((KG_E, E_DIM), jnp.float32),
    )(name, p, p, rs, rs)


NC = 2
NS = 16
L = 16
CH = 128
NCHUNK = N_EDGES // CH
_BASE_CH = NCHUNK // (NC * NS)
_EXTRA = NCHUNK - _BASE_CH * NC * NS

_MESH = plsc.VectorSubcoreMesh(core_axis_name="c", subcore_axis_name="s")


def _edge_sc_body(sl_hbm, sr_hbm, e3_hbm, emb_hbm,
                  zza_hbm, zzb_hbm,
                  out_hbm, ors_hbm,
                  *scr):
    (e30, il0, ir0, slv0, srv0, att0, rw0,
     e31, il1, ir1, slv1, srv1, att1, rw1,
     acc, acc_rs,
     si0, sg0, sro0, sw0, si1, sg1, sro1, sw1) = scr
    bufs = ((e30, il0, ir0, slv0, srv0, att0, rw0,
             si0, sg0, sro0, sw0),
            (e31, il1, ir1, slv1, srv1, att1, rw1,
             si1, sg1, sro1, sw1))
    c = lax.axis_index("c")
    s = lax.axis_index("s")
    wid = s * NC + c

    pltpu.sync_copy(zza_hbm.at[pl.ds(0, 624)], acc.at[pl.ds(s * 624, 624)])
    pltpu.sync_copy(zzb_hbm.at[pl.ds(0, 624)], acc_rs.at[pl.ds(s * 624, 624)])

    @pl.when(s == 0)
    def _():
        pltpu.sync_copy(zza_hbm.at[pl.ds(624, 16)], acc.at[pl.ds(9984, 16)])
        pltpu.sync_copy(zzb_hbm.at[pl.ds(624, 16)], acc_rs.at[pl.ds(9984, 16)])

    plsc.subcore_barrier()

    ch0 = wid * _BASE_CH + jnp.minimum(wid, _EXTRA)

    def fire_ld(ci, b):
        return pltpu.async_copy(e3_hbm.at[ch0 + ci], b[0], b[7])

    def fire_gathers(b):
        e3v, ilv, irv = b[0], b[1], b[2]
        for i in range(CH // L):
            sl16 = e3v[0, pl.ds(i * L, L)]
            dl16 = e3v[1, pl.ds(i * L, L)]
            rl16 = e3v[2, pl.ds(i * L, L)]
            rhi = lax.shift_right_logical(rl16, 7) * (KG_E * 128)
            rlo = lax.bitwise_and(rl16, 127) + rhi
            ilv[0, pl.ds(i * L, L)] = sl16 * 128 + rlo
            irv[0, pl.ds(i * L, L)] = dl16 * 128 + rlo
        rg = pltpu.async_copy(emb_hbm.at[e3v.at[1]], b[6], b[9])
        g1 = pltpu.async_copy(sl_hbm.at[ilv.at[0]], b[3], b[8])
        g2 = pltpu.async_copy(sr_hbm.at[irv.at[0]], b[4], b[8])
        return rg, g1, g2

    def att_compute(b):
        slvv, srvv, attv = b[3], b[4], b[5]
        for i in range(CH // L):
            x = slvv[pl.ds(i * L, L)] + srvv[pl.ds(i * L, L)]
            xl = jnp.where(x > 0, x, ALPHA * x)
            attv[pl.ds(i * L, L)] = jnp.exp(-xl)

    def scale(b):
        attv, rowsv = b[5], b[6]

        def scale_grp(g2, _):
            att16 = attv[pl.ds(g2 * L, L)]
            for k in range(L):
                r = g2 * L + k
                a = att16[k]
                for jb in range(E_DIM // L):
                    rowsv[r, pl.ds(jb * L, L)] = rowsv[r, pl.ds(jb * L, L)] * a
            return 0

        lax.fori_loop(0, CH // L, scale_grp, 0)

    def fire_scatter(b):
        e3v, attv, rowsv = b[0], b[5], b[6]
        return [pltpu.async_copy(rowsv, acc.at[e3v.at[0]], b[10], add=True),
                pltpu.async_copy(attv, acc_rs.at[e3v.at[0]], b[10], add=True)]

    def pair(t, carry):
        ciA = t * 2
        ciB = ciA + 1
        A, B = bufs
        ldA = fire_ld(ciA, A)
        ldB = fire_ld(ciB, B)
        ldA.wait()
        rgA, gA1, gA2 = fire_gathers(A)
        ldB.wait()
        rgB, gB1, gB2 = fire_gathers(B)
        gA1.wait()
        gA2.wait()
        att_compute(A)
        rgA.wait()
        scale(A)
        scA = fire_scatter(A)
        gB1.wait()
        gB2.wait()
        att_compute(B)
        rgB.wait()
        scale(B)
        scB = fire_scatter(B)
        for cp in scA:
            cp.wait()
        for cp in scB:
            cp.wait()
        return 0

    lax.fori_loop(0, _BASE_CH // 2, pair, 0)

    @pl.when(wid < _EXTRA)
    def _():
        A = bufs[0]
        fire_ld(_BASE_CH, A).wait()
        rgA, gA1, gA2 = fire_gathers(A)
        gA1.wait()
        gA2.wait()
        att_compute(A)
        rgA.wait()
        scale(A)
        for cp in fire_scatter(A):
            cp.wait()

    plsc.subcore_barrier()

    pltpu.sync_copy(acc.at[pl.ds(s * 624, 624)],
                    out_hbm.at[pl.ds(c * KG_E + s * 624, 624)])
    pltpu.sync_copy(acc_rs.at[pl.ds(s * 624, 624)],
                    ors_hbm.at[pl.ds(c * KG_E + s * 624, 624)])

    @pl.when(s == 0)
    def _():
        pltpu.sync_copy(acc.at[pl.ds(9984, 16)],
                        out_hbm.at[pl.ds(c * KG_E + 9984, 16)])
        pltpu.sync_copy(acc_rs.at[pl.ds(9984, 16)],
                        ors_hbm.at[pl.ds(c * KG_E + 9984, 16)])


_edge_sc = pl.kernel(
    _edge_sc_body,
    out_type=[
        jax.ShapeDtypeStruct((NC * KG_E, E_DIM), jnp.float32),
        jax.ShapeDtypeStruct((NC * KG_E,), jnp.float32),
    ],
    mesh=_MESH,
    compiler_params=pltpu.CompilerParams(use_tc_tiling_on_sc=False),
    scratch_types=(
        2 * [
            pltpu.VMEM((3, 128), jnp.int32),
            pltpu.VMEM((1, 128), jnp.int32),
            pltpu.VMEM((1, 128), jnp.int32),
            pltpu.VMEM((CH,), jnp.float32),
            pltpu.VMEM((CH,), jnp.float32),
            pltpu.VMEM((CH,), jnp.float32),
            pltpu.VMEM((CH, E_DIM), jnp.float32),
        ]
        + [
            pltpu.VMEM_SHARED((KG_E, E_DIM), jnp.float32),
            pltpu.VMEM_SHARED((KG_E,), jnp.float32),
        ]
        + 8 * [pltpu.SemaphoreType.DMA]
    ),
)


def _edge_stage(sl3, sr3, e3, emb, zza, zzb):
    p, rs = _edge_sc(sl3.reshape(-1), sr3.reshape(-1), e3, emb, zza, zzb)
    return p, rs.reshape(NC * KG_E, 1)


def kernel(kg_name_embed, eer_adj_index, eer_adj_data, r_head, r_tail,
           kg_name_w, kg_name_b, w_R_Left, w_R_Right, w_atten_r):
    b2 = kg_name_b.reshape(1, E_DIM)
    al = w_atten_r[:E_DIM, 0].reshape(1, E_DIM)
    ar = w_atten_r[E_DIM:, 0].reshape(1, E_DIM)
    e3 = jnp.stack([eer_adj_index[0].reshape(-1, 128),
                    eer_adj_index[1].reshape(-1, 128),
                    eer_adj_data.reshape(-1, 128)], axis=1)
    zza = jnp.zeros((640, E_DIM), jnp.float32)
    zzb = jnp.zeros((640,), jnp.float32)
    pad = ((0, KG_RP - KG_R), (0, 0))

    name, l1, r1 = _k1(kg_name_embed, kg_name_w, b2, w_R_Left, w_R_Right)
    u1, v1 = _ku(r_head, r_tail, l1, r1, al, ar)
    sl1, sr1 = _ks(name, jnp.pad(u1, pad), jnp.pad(v1, pad))
    p1, rs1x = _edge_stage(sl1, sr1, e3, name, zza, zzb)
    g1, l2, r2 = _kcomb(name, p1, rs1x, w_R_Left, w_R_Right)
    u2, v2 = _ku(r_head, r_tail, l2, r2, al, ar)
    sl2, sr2 = _ks(g1, jnp.pad(u2, pad), jnp.pad(v2, pad))
    p2, rs2x = _edge_stage(sl2, sr2, e3, g1, zza, zzb)
    return _kcomb_final(name, p2, rs2x)


# speedup vs baseline: 10.6731x; 1.0029x over previous
import functools

import jax
import jax.numpy as jnp
from jax import lax
from jax.experimental import pallas as pl
from jax.experimental.pallas import tpu as pltpu
from jax.experimental.pallas import tpu_sc as plsc

KG_E = 10000
KG_R = 1000
KG_RP = 1024
E_DIM = 128
N_EDGES = 320000
ALPHA = 0.2
BETA1 = 0.3
RHI = KG_RP // 128


def _k1_body(x_ref, w_ref, b_ref, wl_ref, wr_ref, o_ref, l_ref, r_ref):
    y = jnp.dot(x_ref[...], w_ref[...], preferred_element_type=jnp.float32)
    y = y + b_ref[...]
    o_ref[...] = y
    l_ref[...] = jnp.dot(y, wl_ref[...], preferred_element_type=jnp.float32)
    r_ref[...] = jnp.dot(y, wr_ref[...], preferred_element_type=jnp.float32)


def _k1(kg, w, b2, wl, wr):
    blk = 1000
    grid = KG_E // blk
    return pl.pallas_call(
        _k1_body,
        grid=(grid,),
        in_specs=[
            pl.BlockSpec((blk, 300), lambda i: (i, 0)),
            pl.BlockSpec((300, E_DIM), lambda i: (0, 0)),
            pl.BlockSpec((1, E_DIM), lambda i: (0, 0)),
            pl.BlockSpec((E_DIM, E_DIM), lambda i: (0, 0)),
            pl.BlockSpec((E_DIM, E_DIM), lambda i: (0, 0)),
        ],
        out_specs=[
            pl.BlockSpec((blk, E_DIM), lambda i: (i, 0)),
            pl.BlockSpec((blk, E_DIM), lambda i: (i, 0)),
            pl.BlockSpec((blk, E_DIM), lambda i: (i, 0)),
        ],
        out_shape=[
            jax.ShapeDtypeStruct((KG_E, E_DIM), jnp.float32),
            jax.ShapeDtypeStruct((KG_E, E_DIM), jnp.float32),
            jax.ShapeDtypeStruct((KG_E, E_DIM), jnp.float32),
        ],
    )(kg, w, b2, wl, wr)


def _ku_body(rh_ref, rt_ref, le_ref, re_ref, al_ref, ar_ref, u_ref, v_ref):
    rh = rh_ref[...]
    rt = rt_ref[...]
    hs = jnp.sum(rh, axis=1, keepdims=True)
    ts = jnp.sum(rt, axis=1, keepdims=True)
    hinv = jnp.where(hs == 0, 0.0, 1.0 / hs)
    tinv = jnp.where(ts == 0, 0.0, 1.0 / ts)
    lr = jnp.dot(rh, le_ref[...], preferred_element_type=jnp.float32) * hinv
    rr = jnp.dot(rt, re_ref[...], preferred_element_type=jnp.float32) * tinv
    u_ref[...] = jax.nn.relu(lr) * al_ref[...]
    v_ref[...] = jax.nn.relu(rr) * ar_ref[...]


def _ku(r_head, r_tail, le, re, al, ar):
    blk = 200
    grid = KG_R // blk
    return pl.pallas_call(
        _ku_body,
        grid=(grid,),
        in_specs=[
            pl.BlockSpec((blk, KG_E), lambda i: (i, 0)),
            pl.BlockSpec((blk, KG_E), lambda i: (i, 0)),
            pl.BlockSpec((KG_E, E_DIM), lambda i: (0, 0)),
            pl.BlockSpec((KG_E, E_DIM), lambda i: (0, 0)),
            pl.BlockSpec((1, E_DIM), lambda i: (0, 0)),
            pl.BlockSpec((1, E_DIM), lambda i: (0, 0)),
        ],
        out_specs=[
            pl.BlockSpec((blk, E_DIM), lambda i: (i, 0)),
            pl.BlockSpec((blk, E_DIM), lambda i: (i, 0)),
        ],
        out_shape=[
            jax.ShapeDtypeStruct((KG_R, E_DIM), jnp.float32),
            jax.ShapeDtypeStruct((KG_R, E_DIM), jnp.float32),
        ],
    )(r_head, r_tail, le, re, al, ar)


def _ks_body(e_ref, u_ref, v_ref, sl_ref, sr_ref):
    e = e_ref[...].astype(jnp.bfloat16)
    dn = (((1,), (1,)), ((), ()))
    sl_ref[...] = lax.dot_general(e, u_ref[...].astype(jnp.bfloat16), dn,
                                  preferred_element_type=jnp.float32)[None]
    sr_ref[...] = lax.dot_general(e, v_ref[...].astype(jnp.bfloat16), dn,
                                  preferred_element_type=jnp.float32)[None]


def _ks(ent, u_pad, v_pad):
    blk = 2000
    grid = KG_E // blk
    return pl.pallas_call(
        _ks_body,
        grid=(grid, RHI),
        in_specs=[
            pl.BlockSpec((blk, E_DIM), lambda i, j: (i, 0)),
            pl.BlockSpec((128, E_DIM), lambda i, j: (j, 0)),
            pl.BlockSpec((128, E_DIM), lambda i, j: (j, 0)),
        ],
        out_specs=[
            pl.BlockSpec((1, blk, 128), lambda i, j: (j, i, 0)),
            pl.BlockSpec((1, blk, 128), lambda i, j: (j, i, 0)),
        ],
        out_shape=[
            jax.ShapeDtypeStruct((RHI, KG_E, 128), jnp.float32),
            jax.ShapeDtypeStruct((RHI, KG_E, 128), jnp.float32),
        ],
    )(ent, u_pad, v_pad)


def _kcomb_body(n_ref, p0_ref, p1_ref, rs0_ref, rs1_ref, wl_ref, wr_ref,
                g_ref, l_ref, r_ref):
    p = p0_ref[...] + p1_ref[...]
    rs = rs0_ref[...] + rs1_ref[...]
    inv = jnp.where(rs == 0, 0.0, 1.0 / rs)
    e_att = jax.nn.relu(p * inv)
    g = n_ref[...] + BETA1 * e_att
    g_ref[...] = g
    l_ref[...] = jnp.dot(g, wl_ref[...], preferred_element_type=jnp.float32)
    r_ref[...] = jnp.dot(g, wr_ref[...], preferred_element_type=jnp.float32)


def _kcomb(name, p, rs, wl, wr):
    blk = 1000
    grid = KG_E // blk
    nb = grid
    return pl.pallas_call(
        _kcomb_body,
        grid=(grid,),
        in_specs=[
            pl.BlockSpec((blk, E_DIM), lambda i: (i, 0)),
            pl.BlockSpec((blk, E_DIM), lambda i: (i, 0)),
            pl.BlockSpec((blk, E_DIM), lambda i: (i + nb, 0)),
            pl.BlockSpec((blk, 1), lambda i: (i, 0)),
            pl.BlockSpec((blk, 1), lambda i: (i + nb, 0)),
            pl.BlockSpec((E_DIM, E_DIM), lambda i: (0, 0)),
            pl.BlockSpec((E_DIM, E_DIM), lambda i: (0, 0)),
        ],
        out_specs=[
            pl.BlockSpec((blk, E_DIM), lambda i: (i, 0)),
            pl.BlockSpec((blk, E_DIM), lambda i: (i, 0)),
            pl.BlockSpec((blk, E_DIM), lambda i: (i, 0)),
        ],
        out_shape=[
            jax.ShapeDtypeStruct((KG_E, E_DIM), jnp.float32),
            jax.ShapeDtypeStruct((KG_E, E_DIM), jnp.float32),
            jax.ShapeDtypeStruct((KG_E, E_DIM), jnp.float32),
        ],
    )(name, p, p, rs, rs, wl, wr)


def _kcomb_final_body(n_ref, p0_ref, p1_ref, rs0_ref, rs1_ref, g_ref):
    p = p0_ref[...] + p1_ref[...]
    rs = rs0_ref[...] + rs1_ref[...]
    inv = jnp.where(rs == 0, 0.0, 1.0 / rs)
    e_att = jax.nn.relu(p * inv)
    g_ref[...] = n_ref[...] + BETA1 * e_att


def _kcomb_final(name, p, rs):
    blk = 1000
    grid = KG_E // blk
    nb = grid
    return pl.pallas_call(
        _kcomb_final_body,
        grid=(grid,),
        in_specs=[
            pl.BlockSpec((blk, E_DIM), lambda i: (i, 0)),
            pl.BlockSpec((blk, E_DIM), lambda i: (i, 0)),
            pl.BlockSpec((blk, E_DIM), lambda i: (i + nb, 0)),
            pl.BlockSpec((blk, 1), lambda i: (i, 0)),
            pl.BlockSpec((blk, 1), lambda i: (i + nb, 0)),
        ],
        out_specs=pl.BlockSpec((blk, E_DIM), lambda i: (i, 0)),
        out_shape=jax.ShapeDtypeStruct((KG_E, E_DIM), jnp.float32),
    )(name, p, p, rs, rs)


NC = 2
NS = 16
L = 16
CH = 128
NCHUNK = N_EDGES // CH
_BASE_CH = NCHUNK // (NC * NS)
_EXTRA = NCHUNK - _BASE_CH * NC * NS

_MESH = plsc.VectorSubcoreMesh(core_axis_name="c", subcore_axis_name="s")


def _edge_sc_body(sl_hbm, sr_hbm, e3_hbm, emb_hbm,
                  zza_hbm, zzb_hbm,
                  out_hbm, ors_hbm,
                  *scr):
    (e30, il0, ir0, slv0, srv0, att0, rw0,
     e31, il1, ir1, slv1, srv1, att1, rw1,
     acc, acc_rs,
     si0, sg0, sro0, sw0, si1, sg1, sro1, sw1) = scr
    bufs = ((e30, il0, ir0, slv0, srv0, att0, rw0,
             si0, sg0, sro0, sw0),
            (e31, il1, ir1, slv1, srv1, att1, rw1,
             si1, sg1, sro1, sw1))
    c = lax.axis_index("c")
    s = lax.axis_index("s")
    wid = s * NC + c

    pltpu.sync_copy(zza_hbm.at[pl.ds(0, 624)], acc.at[pl.ds(s * 624, 624)])
    pltpu.sync_copy(zzb_hbm.at[pl.ds(0, 624)], acc_rs.at[pl.ds(s * 624, 624)])

    @pl.when(s == 0)
    def _():
        pltpu.sync_copy(zza_hbm.at[pl.ds(624, 16)], acc.at[pl.ds(9984, 16)])
        pltpu.sync_copy(zzb_hbm.at[pl.ds(624, 16)], acc_rs.at[pl.ds(9984, 16)])

    plsc.subcore_barrier()

    ch0 = wid * _BASE_CH + jnp.minimum(wid, _EXTRA)

    def fire_ld(ci, b):
        return pltpu.async_copy(e3_hbm.at[ch0 + ci], b[0], b[7])

    def fire_gathers(b):
        e3v, ilv, irv = b[0], b[1], b[2]
        rg = pltpu.async_copy(emb_hbm.at[e3v.at[1]], b[6], b[9])
        for i in range(CH // L):
            sl16 = e3v[0, pl.ds(i * L, L)]
            dl16 = e3v[1, pl.ds(i * L, L)]
            rl16 = e3v[2, pl.ds(i * L, L)]
            rhi = lax.shift_right_logical(rl16, 7) * (KG_E * 128)
            rlo = lax.bitwise_and(rl16, 127) + rhi
            ilv[0, pl.ds(i * L, L)] = sl16 * 128 + rlo
            irv[0, pl.ds(i * L, L)] = dl16 * 128 + rlo
        g1 = pltpu.async_copy(sl_hbm.at[ilv.at[0]], b[3], b[8])
        g2 = pltpu.async_copy(sr_hbm.at[irv.at[0]], b[4], b[8])
        return rg, g1, g2

    def att_compute(b):
        slvv, srvv, attv = b[3], b[4], b[5]
        for i in range(CH // L):
            x = slvv[pl.ds(i * L, L)] + srvv[pl.ds(i * L, L)]
            xl = jnp.where(x > 0, x, ALPHA * x)
            attv[pl.ds(i * L, L)] = jnp.exp(-xl)

    def scale(b):
        attv, rowsv = b[5], b[6]

        def scale_grp(g2, _):
            att16 = attv[pl.ds(g2 * L, L)]
            for k in range(L):
                r = g2 * L + k
                a = att16[k]
                for jb in range(E_DIM // L):
                    rowsv[r, pl.ds(jb * L, L)] = rowsv[r, pl.ds(jb * L, L)] * a
            return 0

        lax.fori_loop(0, CH // L, scale_grp, 0)

    def fire_scatter(b):
        e3v, attv, rowsv = b[0], b[5], b[6]
        return [pltpu.async_copy(rowsv, acc.at[e3v.at[0]], b[10], add=True),
                pltpu.async_copy(attv, acc_rs.at[e3v.at[0]], b[10], add=True)]

    def pair(t, carry):
        ciA = t * 2
        ciB = ciA + 1
        A, B = bufs
        ldA = fire_ld(ciA, A)
        ldB = fire_ld(ciB, B)
        ldA.wait()
        rgA, gA1, gA2 = fire_gathers(A)
        ldB.wait()
        rgB, gB1, gB2 = fire_gathers(B)
        gA1.wait()
        gA2.wait()
        att_compute(A)
        rgA.wait()
        scale(A)
        scA = fire_scatter(A)
        gB1.wait()
        gB2.wait()
        att_compute(B)
        rgB.wait()
        scale(B)
        scB = fire_scatter(B)
        for cp in scA:
            cp.wait()
        for cp in scB:
            cp.wait()
        return 0

    lax.fori_loop(0, _BASE_CH // 2, pair, 0)

    @pl.when(wid < _EXTRA)
    def _():
        A = bufs[0]
        fire_ld(_BASE_CH, A).wait()
        rgA, gA1, gA2 = fire_gathers(A)
        gA1.wait()
        gA2.wait()
        att_compute(A)
        rgA.wait()
        scale(A)
        for cp in fire_scatter(A):
            cp.wait()

    plsc.subcore_barrier()

    pltpu.sync_copy(acc.at[pl.ds(s * 624, 624)],
                    out_hbm.at[pl.ds(c * KG_E + s * 624, 624)])
    pltpu.sync_copy(acc_rs.at[pl.ds(s * 624, 624)],
                    ors_hbm.at[pl.ds(c * KG_E + s * 624, 624)])

    @pl.when(s == 0)
    def _():
        pltpu.sync_copy(acc.at[pl.ds(9984, 16)],
                        out_hbm.at[pl.ds(c * KG_E + 9984, 16)])
        pltpu.sync_copy(acc_rs.at[pl.ds(9984, 16)],
                        ors_hbm.at[pl.ds(c * KG_E + 9984, 16)])


_edge_sc = pl.kernel(
    _edge_sc_body,
    out_type=[
        jax.ShapeDtypeStruct((NC * KG_E, E_DIM), jnp.float32),
        jax.ShapeDtypeStruct((NC * KG_E,), jnp.float32),
    ],
    mesh=_MESH,
    compiler_params=pltpu.CompilerParams(use_tc_tiling_on_sc=False),
    scratch_types=(
        2 * [
            pltpu.VMEM((3, 128), jnp.int32),
            pltpu.VMEM((1, 128), jnp.int32),
            pltpu.VMEM((1, 128), jnp.int32),
            pltpu.VMEM((CH,), jnp.float32),
            pltpu.VMEM((CH,), jnp.float32),
            pltpu.VMEM((CH,), jnp.float32),
            pltpu.VMEM((CH, E_DIM), jnp.float32),
        ]
        + [
            pltpu.VMEM_SHARED((KG_E, E_DIM), jnp.float32),
            pltpu.VMEM_SHARED((KG_E,), jnp.float32),
        ]
        + 8 * [pltpu.SemaphoreType.DMA]
    ),
)


def _edge_stage(sl3, sr3, e3, emb, zza, zzb):
    p, rs = _edge_sc(sl3.reshape(-1), sr3.reshape(-1), e3, emb, zza, zzb)
    return p, rs.reshape(NC * KG_E, 1)


def kernel(kg_name_embed, eer_adj_index, eer_adj_data, r_head, r_tail,
           kg_name_w, kg_name_b, w_R_Left, w_R_Right, w_atten_r):
    b2 = kg_name_b.reshape(1, E_DIM)
    al = w_atten_r[:E_DIM, 0].reshape(1, E_DIM)
    ar = w_atten_r[E_DIM:, 0].reshape(1, E_DIM)
    e3 = jnp.stack([eer_adj_index[0].reshape(-1, 128),
                    eer_adj_index[1].reshape(-1, 128),
                    eer_adj_data.reshape(-1, 128)], axis=1)
    zza = jnp.zeros((640, E_DIM), jnp.float32)
    zzb = jnp.zeros((640,), jnp.float32)
    pad = ((0, KG_RP - KG_R), (0, 0))

    name, l1, r1 = _k1(kg_name_embed, kg_name_w, b2, w_R_Left, w_R_Right)
    u1, v1 = _ku(r_head, r_tail, l1, r1, al, ar)
    sl1, sr1 = _ks(name, jnp.pad(u1, pad), jnp.pad(v1, pad))
    p1, rs1x = _edge_stage(sl1, sr1, e3, name, zza, zzb)
    g1, l2, r2 = _kcomb(name, p1, rs1x, w_R_Left, w_R_Right)
    u2, v2 = _ku(r_head, r_tail, l2, r2, al, ar)
    sl2, sr2 = _ks(g1, jnp.pad(u2, pad), jnp.pad(v2, pad))
    p2, rs2x = _edge_stage(sl2, sr2, e3, g1, zza, zzb)
    return _kcomb_final(name, p2, rs2x)
